# Initial kernel scaffold; baseline (speedup 1.0000x reference)
import jax
import jax.numpy as jnp
from jax.experimental import pallas as pl


def kernel(x, edge_index, edge_attr, agent_features, params):
    raise NotImplementedError("write your pallas kernel here")


# speedup vs baseline: 1.8846x; 1.8846x over previous
import functools

import jax
import jax.numpy as jnp
from jax import lax
from jax.experimental import pallas as pl
from jax.experimental.pallas import tpu as pltpu

N_NODES = 10000
NPAD = 10240
N_EDGES = 320000
HID = 128
ROW_BLK = 2048
N_BLKS = NPAD // ROW_BLK
UW = 144

_NEG_SLOPE = 0.2


def _leaky(x):
    return jnp.maximum(x, _NEG_SLOPE * x)


def _ln_rows(x, g, b, eps=1e-5):
    mu = jnp.mean(x, axis=-1, keepdims=True)
    var = jnp.mean((x - mu) ** 2, axis=-1, keepdims=True)
    return (x - mu) * jax.lax.rsqrt(var + eps) * g + b


def _pre_body(x_ref, wne_ref, bne_ref, wvn_ref, bvn_ref, h0_ref, vn_ref, acc):
    i = pl.program_id(0)
    h0 = jax.nn.relu(
        jnp.dot(x_ref[...], wne_ref[...], preferred_element_type=jnp.float32)
        + bne_ref[...]
    )
    h0_ref[...] = h0

    @pl.when(i == 0)
    def _():
        acc[...] = jnp.zeros_like(acc)

    ids = i * ROW_BLK + jax.lax.broadcasted_iota(jnp.int32, (ROW_BLK, 1), 0)
    acc[...] += jnp.sum(jnp.where(ids < N_NODES, h0, 0.0), axis=0,
                        keepdims=True)

    @pl.when(i == N_BLKS - 1)
    def _():
        mean = acc[...] / float(N_NODES)
        vn_ref[...] = jnp.tanh(
            jnp.dot(mean, wvn_ref[...], preferred_element_type=jnp.float32)
            + bvn_ref[...]
        )


def _k_pre(x, W_ne, b_ne, W_vn, b_vn):
    return pl.pallas_call(
        _pre_body,
        grid=(N_BLKS,),
        in_specs=[
            pl.BlockSpec((ROW_BLK, HID), lambda i: (i, 0)),
            pl.BlockSpec((HID, HID), lambda i: (0, 0)),
            pl.BlockSpec((1, HID), lambda i: (0, 0)),
            pl.BlockSpec((HID, HID), lambda i: (0, 0)),
            pl.BlockSpec((1, HID), lambda i: (0, 0)),
        ],
        out_specs=[
            pl.BlockSpec((ROW_BLK, HID), lambda i: (i, 0)),
            pl.BlockSpec((1, HID), lambda i: (0, 0)),
        ],
        out_shape=[
            jax.ShapeDtypeStruct((NPAD, HID), jnp.float32),
            jax.ShapeDtypeStruct((1, HID), jnp.float32),
        ],
        scratch_shapes=[pltpu.VMEM((1, HID), jnp.float32)],
    )(x, W_ne, b_ne, W_vn, b_vn)


_CE_BLK = 12800


def _ce_body(ea_ref, wem_ref, ce_ref):
    ea = ea_ref[...]
    wem = wem_ref[...]
    ce = jnp.dot(ea, wem, preferred_element_type=jnp.float32)
    ce_ref[...] = ce.T


def _k_ce(edge_attr, weM):
    return pl.pallas_call(
        _ce_body,
        grid=(N_EDGES // _CE_BLK,),
        in_specs=[
            pl.BlockSpec((_CE_BLK, 3), lambda i: (i, 0)),
            pl.BlockSpec((3, 8), lambda i: (0, 0)),
        ],
        out_specs=pl.BlockSpec((8, _CE_BLK), lambda i: (0, i)),
        out_shape=jax.ShapeDtypeStruct((8, N_EDGES), jnp.float32),
    )(edge_attr, weM)


def _att_body(h_ref, hv_ref, w_ref, apack_ref, wx_ref, scal_ref, wxv_ref):
    i = pl.program_id(0)
    ap = apack_ref[...]
    cv = apack_ref[2, 0]
    wx = jnp.dot(h_ref[...], w_ref[...], preferred_element_type=jnp.float32)
    wx_ref[...] = wx
    wxv = jnp.dot(hv_ref[...], w_ref[...], preferred_element_type=jnp.float32)
    asv = jnp.sum(wxv * ap[0:1, :])
    adv = jnp.sum(wxv * ap[1:2, :])
    sc2 = jax.lax.dot_general(ap, wx, (((1,), (1,)), ((), ())),
                              preferred_element_type=jnp.float32)
    asrc = sc2[0:1, :]
    adst = sc2[1:2, :]
    lv = _leaky(asv + adst + cv)
    ids = i * ROW_BLK + jax.lax.broadcasted_iota(jnp.int32, asrc.shape, 1)
    l2v = jnp.where(ids < N_NODES, _leaky(asrc + adv + cv), -jnp.inf)
    z = jnp.zeros((4,) + asrc.shape[1:], jnp.float32)
    scal_ref[...] = jnp.concatenate([asrc, adst, lv, l2v, z], axis=0)

    @pl.when(i == 0)
    def _():
        wxv_ref[...] = wxv


def _k_att(h, hv, W, apack):
    return pl.pallas_call(
        _att_body,
        grid=(N_BLKS,),
        in_specs=[
            pl.BlockSpec((ROW_BLK, HID), lambda i: (i, 0)),
            pl.BlockSpec((1, HID), lambda i: (0, 0)),
            pl.BlockSpec((HID, HID), lambda i: (0, 0)),
            pl.BlockSpec((8, HID), lambda i: (0, 0)),
        ],
        out_specs=[
            pl.BlockSpec((ROW_BLK, HID), lambda i: (i, 0)),
            pl.BlockSpec((8, ROW_BLK), lambda i: (0, i)),
            pl.BlockSpec((1, HID), lambda i: (0, 0)),
        ],
        out_shape=[
            jax.ShapeDtypeStruct((NPAD, HID), jnp.float32),
            jax.ShapeDtypeStruct((8, NPAD), jnp.float32),
            jax.ShapeDtypeStruct((1, HID), jnp.float32),
        ],
    )(h, hv, W, apack)


def _post_body(u2_ref, wx_ref, scal_ref, h_ref, hv_ref, wxv_ref, pk_ref,
               hn_ref, hvn_ref, m_sc, s_sc, acc):
    i = pl.program_id(0)
    b = pk_ref[0:1, :]
    ln_g = pk_ref[1:2, :]
    ln_b = pk_ref[2:3, :]
    u = u2_ref[0, :, 0:HID] + u2_ref[1, :, 0:HID]
    sreal = u2_ref[0, :, HID:HID + 1] + u2_ref[1, :, HID:HID + 1]
    agg = (u + wxv_ref[...]) / (sreal + 1.0) + b
    hn = _ln_rows(agg, ln_g, ln_b)
    hn_ref[...] = jax.nn.relu(hn + h_ref[...])

    lb = scal_ref[3:4, :]
    mloc = jnp.max(lb)

    @pl.when(i == 0)
    def _():
        m_sc[0, 0] = -jnp.inf
        s_sc[0, 0] = 0.0
        acc[...] = jnp.zeros_like(acc)

    m_old = m_sc[0, 0]
    m_new = jnp.maximum(m_old, mloc)
    scale = jnp.where(m_old == -jnp.inf, 0.0, jnp.exp(m_old - m_new))
    wts = jnp.exp(lb - m_new)
    acc[...] = acc[...] * scale + jnp.dot(
        wts, wx_ref[...], preferred_element_type=jnp.float32
    )
    s_sc[0, 0] = s_sc[0, 0] * scale + jnp.sum(wts)
    m_sc[0, 0] = m_new

    @pl.when(i == N_BLKS - 1)
    def _():
        aggv = acc[...] / s_sc[0, 0] + b
        hnv = _ln_rows(aggv, ln_g, ln_b)
        hvn_ref[...] = jax.nn.relu(hnv + hv_ref[...])


def _k_post(U2, Wx, SCAL, h, hv, Wxv, pk):
    return pl.pallas_call(
        _post_body,
        grid=(N_BLKS,),
        in_specs=[
            pl.BlockSpec((2, ROW_BLK, UW), lambda i: (0, i, 0)),
            pl.BlockSpec((ROW_BLK, HID), lambda i: (i, 0)),
            pl.BlockSpec((8, ROW_BLK), lambda i: (0, i)),
            pl.BlockSpec((ROW_BLK, HID), lambda i: (i, 0)),
            pl.BlockSpec((1, HID), lambda i: (0, 0)),
            pl.BlockSpec((1, HID), lambda i: (0, 0)),
            pl.BlockSpec((8, HID), lambda i: (0, 0)),
        ],
        out_specs=[
            pl.BlockSpec((ROW_BLK, HID), lambda i: (i, 0)),
            pl.BlockSpec((1, HID), lambda i: (0, 0)),
        ],
        out_shape=[
            jax.ShapeDtypeStruct((NPAD, HID), jnp.float32),
            jax.ShapeDtypeStruct((1, HID), jnp.float32),
        ],
        scratch_shapes=[
            pltpu.SMEM((1, 1), jnp.float32),
            pltpu.SMEM((1, 1), jnp.float32),
            pltpu.VMEM((1, HID), jnp.float32),
        ],
    )(U2, Wx, SCAL, h, hv, Wxv, pk)


def _last_body(h_ref, hv_ref, w_ref, apack_ref, pk_ref,
               hv01_ref, agent_ref,
               wiht_ref, bih_ref, bhh_ref, wop_ref, bop_ref, ln2_ref,
               wrp_ref, brp_ref, wag_ref, bag_ref, wv_ref, bv_ref,
               wa_ref, ba_ref,
               out_ref, m_sc, s_sc, acc):
    i = pl.program_id(0)
    a_src = apack_ref[0:1, :]
    a_dst = apack_ref[1:2, :]
    cv = apack_ref[2, 0]
    wx = jnp.dot(h_ref[...], w_ref[...], preferred_element_type=jnp.float32)
    wxv = jnp.dot(hv_ref[...], w_ref[...], preferred_element_type=jnp.float32)
    adv = jnp.sum(wxv * a_dst)
    asrc = jax.lax.dot_general(a_src, wx, (((1,), (1,)), ((), ())),
                               preferred_element_type=jnp.float32)
    ids = i * ROW_BLK + jax.lax.broadcasted_iota(jnp.int32, asrc.shape, 1)
    l2v = jnp.where(ids < N_NODES, _leaky(asrc + adv + cv), -jnp.inf)
    mloc = jnp.max(l2v)

    @pl.when(i == 0)
    def _():
        m_sc[0, 0] = -jnp.inf
        s_sc[0, 0] = 0.0
        acc[...] = jnp.zeros_like(acc)

    m_old = m_sc[0, 0]
    m_new = jnp.maximum(m_old, mloc)
    scale = jnp.where(m_old == -jnp.inf, 0.0, jnp.exp(m_old - m_new))
    wts = jnp.exp(l2v - m_new)
    acc[...] = acc[...] * scale + jnp.dot(
        wts, wx, preferred_element_type=jnp.float32
    )
    s_sc[0, 0] = s_sc[0, 0] * scale + jnp.sum(wts)
    m_sc[0, 0] = m_new

    @pl.when(i == N_BLKS - 1)
    def _():
        b = pk_ref[0:1, :]
        ln_g = pk_ref[1:2, :]
        ln_b = pk_ref[2:3, :]
        aggv = acc[...] / s_sc[0, 0] + b
        hnv = _ln_rows(aggv, ln_g, ln_b)
        hv3 = jax.nn.relu(hnv + hv_ref[...])
        jk = jnp.concatenate([hv01_ref[0:1], hv01_ref[1:2], hv01_ref[2:3],
                              hv3], axis=1)
        gi = jnp.dot(jk, wiht_ref[...],
                     preferred_element_type=jnp.float32) + bih_ref[...]
        bhh = bhh_ref[...]
        r = jax.nn.sigmoid(gi[:, 0:HID] + bhh[:, 0:HID])
        z = jax.nn.sigmoid(gi[:, HID:2 * HID] + bhh[:, HID:2 * HID])
        n = jnp.tanh(gi[:, 2 * HID:] + r * bhh[:, 2 * HID:])
        hidden = (1.0 - z) * n
        op = jnp.dot(hidden, wop_ref[...],
                     preferred_element_type=jnp.float32) + bop_ref[...]
        rec = jax.nn.relu(_ln_rows(op, ln2_ref[0:1, :], ln2_ref[1:2, :]))
        rec = jnp.dot(rec, wrp_ref[...],
                      preferred_element_type=jnp.float32) + brp_ref[...]
        ag = jax.nn.relu(
            jnp.dot(agent_ref[...], wag_ref[...],
                    preferred_element_type=jnp.float32) + bag_ref[...])
        comb = jnp.concatenate([rec, ag], axis=1)
        value = jnp.dot(comb, wv_ref[...],
                        preferred_element_type=jnp.float32) + bv_ref[...]
        adv_q = jnp.dot(comb, wa_ref[...],
                        preferred_element_type=jnp.float32) + ba_ref[...]
        out_ref[...] = value + (adv_q - jnp.mean(adv_q, axis=1, keepdims=True))


def _k_last(h, hv, W, apack, pk, hv01, agent, wiht, bih, bhh, wop, bop,
            ln2, wrp, brp, wag, bag, wv, bv, wa, ba):
    full = lambda r, c: pl.BlockSpec((r, c), lambda i: (0, 0))
    return pl.pallas_call(
        _last_body,
        grid=(N_BLKS,),
        in_specs=[
            pl.BlockSpec((ROW_BLK, HID), lambda i: (i, 0)),
            full(1, HID), full(HID, HID), full(8, HID), full(8, HID),
            full(4, HID), full(1, 16),
            full(512, 384), full(1, 384), full(1, 384),
            full(HID, HID), full(1, HID), full(2, HID),
            full(HID, 512), full(1, 512), full(16, HID), full(1, HID),
            full(640, 1), full(1, 1), full(640, 8), full(1, 8),
        ],
        out_specs=pl.BlockSpec((1, 8), lambda i: (0, 0)),
        out_shape=jax.ShapeDtypeStruct((1, 8), jnp.float32),
        scratch_shapes=[
            pltpu.SMEM((1, 1), jnp.float32),
            pltpu.SMEM((1, 1), jnp.float32),
            pltpu.VMEM((1, HID), jnp.float32),
        ],
    )(h, hv, W, apack, pk, hv01, agent, wiht, bih, bhh, wop, bop,
      ln2, wrp, brp, wag, bag, wv, bv, wa, ba)


def _edge_phase(Wx, SCAL, src, dst, ce):
    asrc = SCAL[0]
    adst = SCAL[1]
    lv = SCAL[2]
    logit = _leaky(asrc[src] + adst[dst] + ce)
    e = jnp.exp(jnp.minimum(logit - lv[dst], 60.0))
    s = jax.ops.segment_sum(e, dst, num_segments=NPAD)
    U = jax.ops.segment_sum(Wx[src] * e[:, None], dst, num_segments=NPAD)
    U2 = jnp.zeros((2, NPAD, UW), jnp.float32)
    U2 = U2.at[0, :, 0:HID].set(U).at[0, :, HID].set(s)
    return U2


def kernel(x, edge_index, edge_attr, agent_features, params):
    src = edge_index[0].astype(jnp.int32)
    dst = edge_index[1].astype(jnp.int32)
    xp = jnp.pad(x, ((0, NPAD - N_NODES), (0, 0)))

    h, hv = _k_pre(xp, params['W_ne'], params['b_ne'][None, :],
                   params['W_vn'], params['b_vn'][None, :])

    gat = params['gat']
    weM = jnp.stack([p['W_e'] @ p['a_e'] for p in gat], axis=1)
    weM = jnp.pad(weM, ((0, 0), (0, 5)))
    CE = _k_ce(edge_attr, weM)

    def apack_of(p, l):
        cv = 0.5 * weM[0, l]
        return jnp.concatenate([
            p['a_src'][None, :], p['a_dst'][None, :],
            jnp.full((1, HID), cv, jnp.float32),
            jnp.zeros((5, HID), jnp.float32)], axis=0)

    def pk_of(p):
        return jnp.concatenate([
            p['b'][None, :], p['ln_g'][None, :], p['ln_b'][None, :],
            jnp.zeros((5, HID), jnp.float32)], axis=0)

    hvs = [hv]
    for l in range(2):
        p = gat[l]
        Wx, SCAL, Wxv = _k_att(h, hv, p['W'], apack_of(p, l))
        U2 = _edge_phase(Wx, SCAL, src, dst, CE[l])
        h, hv = _k_post(U2, Wx, SCAL, h, hv, Wxv, pk_of(p))
        hvs.append(hv)

    p = gat[2]
    hv01 = jnp.concatenate([hvs[0], hvs[1], hvs[2],
                            jnp.zeros((1, HID), jnp.float32)], axis=0)
    out = _k_last(
        h, hv, p['W'], apack_of(p, 2), pk_of(p), hv01, agent_features,
        params['W_ih'].T, params['b_ih'][None, :], params['b_hh'][None, :],
        params['W_op'], params['b_op'][None, :],
        jnp.stack([params['ln2_g'], params['ln2_b']], axis=0),
        params['W_rp'], params['b_rp'][None, :],
        params['W_ag'], params['b_ag'][None, :],
        params['W_v'], params['b_v'][None, :],
        params['W_a'], params['b_a'][None, :])
    return out


# speedup vs baseline: 24.6827x; 13.0970x over previous
import functools

import jax
import jax.numpy as jnp
from jax import lax
from jax.experimental import pallas as pl
from jax.experimental.pallas import tpu as pltpu
from jax.experimental.pallas import tpu_sc as plsc

N_NODES = 10000
NPAD = 10240
N_EDGES = 320000
HID = 128
ROW_BLK = 2048
N_BLKS = NPAD // ROW_BLK
UW = 144

_NEG_SLOPE = 0.2


def _leaky(x):
    return jnp.maximum(x, _NEG_SLOPE * x)


def _ln_rows(x, g, b, eps=1e-5):
    mu = jnp.mean(x, axis=-1, keepdims=True)
    var = jnp.mean((x - mu) ** 2, axis=-1, keepdims=True)
    return (x - mu) * jax.lax.rsqrt(var + eps) * g + b


def _pre_body(x_ref, wne_ref, bne_ref, wvn_ref, bvn_ref, h0_ref, vn_ref, acc):
    i = pl.program_id(0)
    h0 = jax.nn.relu(
        jnp.dot(x_ref[...], wne_ref[...], preferred_element_type=jnp.float32)
        + bne_ref[...]
    )
    h0_ref[...] = h0

    @pl.when(i == 0)
    def _():
        acc[...] = jnp.zeros_like(acc)

    ids = i * ROW_BLK + jax.lax.broadcasted_iota(jnp.int32, (ROW_BLK, 1), 0)
    acc[...] += jnp.sum(jnp.where(ids < N_NODES, h0, 0.0), axis=0,
                        keepdims=True)

    @pl.when(i == N_BLKS - 1)
    def _():
        mean = acc[...] / float(N_NODES)
        vn_ref[...] = jnp.tanh(
            jnp.dot(mean, wvn_ref[...], preferred_element_type=jnp.float32)
            + bvn_ref[...]
        )


def _k_pre(x, W_ne, b_ne, W_vn, b_vn):
    return pl.pallas_call(
        _pre_body,
        grid=(N_BLKS,),
        in_specs=[
            pl.BlockSpec((ROW_BLK, HID), lambda i: (i, 0)),
            pl.BlockSpec((HID, HID), lambda i: (0, 0)),
            pl.BlockSpec((1, HID), lambda i: (0, 0)),
            pl.BlockSpec((HID, HID), lambda i: (0, 0)),
            pl.BlockSpec((1, HID), lambda i: (0, 0)),
        ],
        out_specs=[
            pl.BlockSpec((ROW_BLK, HID), lambda i: (i, 0)),
            pl.BlockSpec((1, HID), lambda i: (0, 0)),
        ],
        out_shape=[
            jax.ShapeDtypeStruct((NPAD, HID), jnp.float32),
            jax.ShapeDtypeStruct((1, HID), jnp.float32),
        ],
        scratch_shapes=[pltpu.VMEM((1, HID), jnp.float32)],
    )(x, W_ne, b_ne, W_vn, b_vn)


_CE_BLK = 12800


def _ce_body(ea_ref, wem_ref, ce_ref):
    ea = ea_ref[...]
    wem = wem_ref[...]
    ce = jnp.dot(ea, wem, preferred_element_type=jnp.float32)
    ce_ref[...] = ce.T


def _k_ce(edge_attr, weM):
    return pl.pallas_call(
        _ce_body,
        grid=(N_EDGES // _CE_BLK,),
        in_specs=[
            pl.BlockSpec((_CE_BLK, 3), lambda i: (i, 0)),
            pl.BlockSpec((3, 8), lambda i: (0, 0)),
        ],
        out_specs=pl.BlockSpec((8, _CE_BLK), lambda i: (0, i)),
        out_shape=jax.ShapeDtypeStruct((8, N_EDGES), jnp.float32),
    )(edge_attr, weM)


def _att_body(h_ref, hv_ref, w_ref, apack_ref, apt_ref, wxa_ref, scal_ref,
              wxv_ref):
    i = pl.program_id(0)
    ap = apack_ref[...]
    cv = apack_ref[2, 0]
    wx = jnp.dot(h_ref[...], w_ref[...], preferred_element_type=jnp.float32)
    wxv = jnp.dot(hv_ref[...], w_ref[...], preferred_element_type=jnp.float32)
    asv = jnp.sum(wxv * ap[0:1, :])
    adv = jnp.sum(wxv * ap[1:2, :])
    acol = jnp.dot(wx, apt_ref[...], preferred_element_type=jnp.float32)
    wxa_ref[...] = jnp.concatenate(
        [wx, acol[:, 0:1], jnp.zeros((wx.shape[0], UW - HID - 1),
                                     jnp.float32)], axis=1)
    sc2 = jax.lax.dot_general(ap, wx, (((1,), (1,)), ((), ())),
                              preferred_element_type=jnp.float32)
    asrc = sc2[0:1, :]
    adst = sc2[1:2, :]
    lv = _leaky(asv + adst + cv)
    ids = i * ROW_BLK + jax.lax.broadcasted_iota(jnp.int32, asrc.shape, 1)
    l2v = jnp.where(ids < N_NODES, _leaky(asrc + adv + cv), -jnp.inf)
    z = jnp.zeros((4,) + asrc.shape[1:], jnp.float32)
    scal_ref[...] = jnp.concatenate([asrc, adst, lv, l2v, z], axis=0)

    @pl.when(i == 0)
    def _():
        wxv_ref[...] = wxv


def _k_att(h, hv, W, apack, apackT):
    return pl.pallas_call(
        _att_body,
        grid=(N_BLKS,),
        in_specs=[
            pl.BlockSpec((ROW_BLK, HID), lambda i: (i, 0)),
            pl.BlockSpec((1, HID), lambda i: (0, 0)),
            pl.BlockSpec((HID, HID), lambda i: (0, 0)),
            pl.BlockSpec((8, HID), lambda i: (0, 0)),
            pl.BlockSpec((HID, 1), lambda i: (0, 0)),
        ],
        out_specs=[
            pl.BlockSpec((ROW_BLK, UW), lambda i: (i, 0)),
            pl.BlockSpec((8, ROW_BLK), lambda i: (0, i)),
            pl.BlockSpec((1, HID), lambda i: (0, 0)),
        ],
        out_shape=[
            jax.ShapeDtypeStruct((NPAD, UW), jnp.float32),
            jax.ShapeDtypeStruct((8, NPAD), jnp.float32),
            jax.ShapeDtypeStruct((1, HID), jnp.float32),
        ],
    )(h, hv, W, apack, apackT)


def _post_body(u2_ref, wx_ref, scal_ref, h_ref, hv_ref, wxv_ref, pk_ref,
               hn_ref, hvn_ref, m_sc, s_sc, acc):
    i = pl.program_id(0)
    b = pk_ref[0:1, :]
    ln_g = pk_ref[1:2, :]
    ln_b = pk_ref[2:3, :]
    u = u2_ref[0, :, 0:HID] + u2_ref[1, :, 0:HID]
    sreal = u2_ref[0, :, HID:HID + 1] + u2_ref[1, :, HID:HID + 1]
    agg = (u + wxv_ref[...]) / (sreal + 1.0) + b
    hn = _ln_rows(agg, ln_g, ln_b)
    hn_ref[...] = jax.nn.relu(hn + h_ref[...])

    wx = wx_ref[...][:, 0:HID]
    lb = scal_ref[3:4, :]
    mloc = jnp.max(lb)

    @pl.when(i == 0)
    def _():
        m_sc[0, 0] = -jnp.inf
        s_sc[0, 0] = 0.0
        acc[...] = jnp.zeros_like(acc)

    m_old = m_sc[0, 0]
    m_new = jnp.maximum(m_old, mloc)
    scale = jnp.where(m_old == -jnp.inf, 0.0, jnp.exp(m_old - m_new))
    wts = jnp.exp(lb - m_new)
    acc[...] = acc[...] * scale + jnp.dot(
        wts, wx, preferred_element_type=jnp.float32
    )
    s_sc[0, 0] = s_sc[0, 0] * scale + jnp.sum(wts)
    m_sc[0, 0] = m_new

    @pl.when(i == N_BLKS - 1)
    def _():
        aggv = acc[...] / s_sc[0, 0] + b
        hnv = _ln_rows(aggv, ln_g, ln_b)
        hvn_ref[...] = jax.nn.relu(hnv + hv_ref[...])


def _k_post(U2, Wx, SCAL, h, hv, Wxv, pk):
    return pl.pallas_call(
        _post_body,
        grid=(N_BLKS,),
        in_specs=[
            pl.BlockSpec((2, ROW_BLK, UW), lambda i: (0, i, 0)),
            pl.BlockSpec((ROW_BLK, UW), lambda i: (i, 0)),
            pl.BlockSpec((8, ROW_BLK), lambda i: (0, i)),
            pl.BlockSpec((ROW_BLK, HID), lambda i: (i, 0)),
            pl.BlockSpec((1, HID), lambda i: (0, 0)),
            pl.BlockSpec((1, HID), lambda i: (0, 0)),
            pl.BlockSpec((8, HID), lambda i: (0, 0)),
        ],
        out_specs=[
            pl.BlockSpec((ROW_BLK, HID), lambda i: (i, 0)),
            pl.BlockSpec((1, HID), lambda i: (0, 0)),
        ],
        out_shape=[
            jax.ShapeDtypeStruct((NPAD, HID), jnp.float32),
            jax.ShapeDtypeStruct((1, HID), jnp.float32),
        ],
        scratch_shapes=[
            pltpu.SMEM((1, 1), jnp.float32),
            pltpu.SMEM((1, 1), jnp.float32),
            pltpu.VMEM((1, HID), jnp.float32),
        ],
    )(U2, Wx, SCAL, h, hv, Wxv, pk)


def _last_body(h_ref, hv_ref, w_ref, apack_ref, pk_ref,
               hv01_ref, agent_ref,
               wiht_ref, bih_ref, bhh_ref, wop_ref, bop_ref, ln2_ref,
               wrp_ref, brp_ref, wag_ref, bag_ref, wv_ref, bv_ref,
               wa_ref, ba_ref,
               out_ref, m_sc, s_sc, acc):
    i = pl.program_id(0)
    a_src = apack_ref[0:1, :]
    a_dst = apack_ref[1:2, :]
    cv = apack_ref[2, 0]
    wx = jnp.dot(h_ref[...], w_ref[...], preferred_element_type=jnp.float32)
    wxv = jnp.dot(hv_ref[...], w_ref[...], preferred_element_type=jnp.float32)
    adv = jnp.sum(wxv * a_dst)
    asrc = jax.lax.dot_general(a_src, wx, (((1,), (1,)), ((), ())),
                               preferred_element_type=jnp.float32)
    ids = i * ROW_BLK + jax.lax.broadcasted_iota(jnp.int32, asrc.shape, 1)
    l2v = jnp.where(ids < N_NODES, _leaky(asrc + adv + cv), -jnp.inf)
    mloc = jnp.max(l2v)

    @pl.when(i == 0)
    def _():
        m_sc[0, 0] = -jnp.inf
        s_sc[0, 0] = 0.0
        acc[...] = jnp.zeros_like(acc)

    m_old = m_sc[0, 0]
    m_new = jnp.maximum(m_old, mloc)
    scale = jnp.where(m_old == -jnp.inf, 0.0, jnp.exp(m_old - m_new))
    wts = jnp.exp(l2v - m_new)
    acc[...] = acc[...] * scale + jnp.dot(
        wts, wx, preferred_element_type=jnp.float32
    )
    s_sc[0, 0] = s_sc[0, 0] * scale + jnp.sum(wts)
    m_sc[0, 0] = m_new

    @pl.when(i == N_BLKS - 1)
    def _():
        b = pk_ref[0:1, :]
        ln_g = pk_ref[1:2, :]
        ln_b = pk_ref[2:3, :]
        aggv = acc[...] / s_sc[0, 0] + b
        hnv = _ln_rows(aggv, ln_g, ln_b)
        hv3 = jax.nn.relu(hnv + hv_ref[...])
        jk = jnp.concatenate([hv01_ref[0:1], hv01_ref[1:2], hv01_ref[2:3],
                              hv3], axis=1)
        gi = jnp.dot(jk, wiht_ref[...],
                     preferred_element_type=jnp.float32) + bih_ref[...]
        bhh = bhh_ref[...]
        r = jax.nn.sigmoid(gi[:, 0:HID] + bhh[:, 0:HID])
        z = jax.nn.sigmoid(gi[:, HID:2 * HID] + bhh[:, HID:2 * HID])
        n = jnp.tanh(gi[:, 2 * HID:] + r * bhh[:, 2 * HID:])
        hidden = (1.0 - z) * n
        op = jnp.dot(hidden, wop_ref[...],
                     preferred_element_type=jnp.float32) + bop_ref[...]
        rec = jax.nn.relu(_ln_rows(op, ln2_ref[0:1, :], ln2_ref[1:2, :]))
        rec = jnp.dot(rec, wrp_ref[...],
                      preferred_element_type=jnp.float32) + brp_ref[...]
        ag = jax.nn.relu(
            jnp.dot(agent_ref[...], wag_ref[...],
                    preferred_element_type=jnp.float32) + bag_ref[...])
        comb = jnp.concatenate([rec, ag], axis=1)
        value = jnp.dot(comb, wv_ref[...],
                        preferred_element_type=jnp.float32) + bv_ref[...]
        adv_q = jnp.dot(comb, wa_ref[...],
                        preferred_element_type=jnp.float32) + ba_ref[...]
        out_ref[...] = value + (adv_q - jnp.mean(adv_q, axis=1, keepdims=True))


def _k_last(h, hv, W, apack, pk, hv01, agent, wiht, bih, bhh, wop, bop,
            ln2, wrp, brp, wag, bag, wv, bv, wa, ba):
    full = lambda r, c: pl.BlockSpec((r, c), lambda i: (0, 0))
    return pl.pallas_call(
        _last_body,
        grid=(N_BLKS,),
        in_specs=[
            pl.BlockSpec((ROW_BLK, HID), lambda i: (i, 0)),
            full(1, HID), full(HID, HID), full(8, HID), full(8, HID),
            full(4, HID), full(1, 16),
            full(512, 384), full(1, 384), full(1, 384),
            full(HID, HID), full(1, HID), full(2, HID),
            full(HID, 512), full(1, 512), full(16, HID), full(1, HID),
            full(640, 1), full(1, 1), full(640, 8), full(1, 8),
        ],
        out_specs=pl.BlockSpec((1, 8), lambda i: (0, 0)),
        out_shape=jax.ShapeDtypeStruct((1, 8), jnp.float32),
        scratch_shapes=[
            pltpu.SMEM((1, 1), jnp.float32),
            pltpu.SMEM((1, 1), jnp.float32),
            pltpu.VMEM((1, HID), jnp.float32),
        ],
    )(h, hv, W, apack, pk, hv01, agent, wiht, bih, bhh, wop, bop,
      ln2, wrp, brp, wag, bag, wv, bv, wa, ba)


_NC = 2
_NS = 16
_NW = _NC * _NS
_EPW = N_EDGES // _NW
_EB = 80
_EBLKS = _EPW // _EB
_RPT = NPAD // _NS


def _sc_edge_body(wxa_hbm, adst_hbm, lv_hbm, src_hbm, dst_hbm,
                  ce_hbm, out_hbm,
                  adst_v, lv_v, src_v, dst_v, ce_v, e_v,
                  srows_v, u_sh, sem):
    cid = lax.axis_index("c")
    sid = lax.axis_index("s")
    wid = sid * _NC + cid
    pltpu.sync_copy(adst_hbm.at[pl.ds(0, N_NODES)], adst_v)
    pltpu.sync_copy(lv_hbm.at[pl.ds(0, N_NODES)], lv_v)

    zero16 = jnp.zeros((16,), jnp.float32)

    def zrow(j, c):
        for k in range(UW // 16):
            srows_v[j, pl.ds(16 * k, 16)] = zero16
        return c

    lax.fori_loop(0, _EB, zrow, 0)
    rbase = sid * _RPT

    def zcp(j, c):
        pltpu.sync_copy(srows_v, u_sh.at[pl.ds(rbase + j * _EB, _EB)])
        return c

    lax.fori_loop(0, _RPT // _EB, zcp, 0)
    plsc.subcore_barrier()

    onehot = jnp.where(lax.iota(jnp.int32, 16) == 0, 1.0, 0.0)
    col128 = jnp.full((16,), HID, jnp.int32)
    iota16 = lax.iota(jnp.int32, 16)
    ebase = wid * _EPW

    def blk(g, c):
        base = ebase + g * _EB
        pltpu.sync_copy(src_hbm.at[pl.ds(base, _EB)], src_v)
        pltpu.sync_copy(dst_hbm.at[pl.ds(base, _EB)], dst_v)
        pltpu.sync_copy(ce_hbm.at[pl.ds(base, _EB)], ce_v)
        pltpu.async_copy(wxa_hbm.at[src_v], srows_v, sem).wait()
        for j in range(_EB // 16):
            d16 = dst_v[pl.ds(16 * j, 16)]
            av = plsc.load_gather(srows_v, [iota16 + (16 * j), col128])
            bv = plsc.load_gather(adst_v, [d16])
            cv0 = plsc.load_gather(lv_v, [d16])
            t = av + bv + ce_v[pl.ds(16 * j, 16)]
            lg = jnp.maximum(t, _NEG_SLOPE * t)
            e_v[pl.ds(16 * j, 16)] = jnp.exp(jnp.minimum(lg - cv0, 60.0))

        def srow(j, c2):
            ev = plsc.load_gather(e_v, [jnp.full((16,), j, jnp.int32)])
            for k in range(HID // 16):
                srows_v[j, pl.ds(16 * k, 16)] = srows_v[j, pl.ds(16 * k, 16)] * ev
            srows_v[j, pl.ds(HID, 16)] = ev * onehot
            return c2

        lax.fori_loop(0, _EB, srow, 0)
        pltpu.sync_copy(srows_v, u_sh.at[dst_v], add=True)
        return c

    lax.fori_loop(0, _EBLKS, blk, 0)
    plsc.subcore_barrier()
    pltpu.sync_copy(u_sh.at[pl.ds(rbase, _RPT)],
                    out_hbm.at[cid, pl.ds(rbase, _RPT)])


@functools.partial(
    pl.kernel,
    mesh=plsc.VectorSubcoreMesh(core_axis_name="c", subcore_axis_name="s"),
    out_type=jax.ShapeDtypeStruct((2, NPAD, UW), jnp.float32),
    compiler_params=pltpu.CompilerParams(needs_layout_passes=False,
                                         use_tc_tiling_on_sc=False),
    scratch_types=[
        pltpu.VMEM((N_NODES,), jnp.float32),
        pltpu.VMEM((N_NODES,), jnp.float32),
        pltpu.VMEM((_EB,), jnp.int32),
        pltpu.VMEM((_EB,), jnp.int32),
        pltpu.VMEM((_EB,), jnp.float32),
        pltpu.VMEM((_EB,), jnp.float32),
        pltpu.VMEM((_EB, UW), jnp.float32),
        pltpu.VMEM_SHARED((NPAD, UW), jnp.float32),
        pltpu.SemaphoreType.DMA,
    ],
)
def _k_sc(wxa_hbm, adst_hbm, lv_hbm, src_hbm, dst_hbm, ce_hbm,
          out_hbm, *rest):
    _sc_edge_body(wxa_hbm, adst_hbm, lv_hbm, src_hbm, dst_hbm,
                  ce_hbm, out_hbm, *rest)


def _edge_phase(WxA, SCAL, src, dst, ce):
    return _k_sc(WxA, SCAL[1], SCAL[2], src, dst, ce)


def kernel(x, edge_index, edge_attr, agent_features, params):
    src = edge_index[0].astype(jnp.int32)
    dst = edge_index[1].astype(jnp.int32)
    xp = jnp.pad(x, ((0, NPAD - N_NODES), (0, 0)))

    h, hv = _k_pre(xp, params['W_ne'], params['b_ne'][None, :],
                   params['W_vn'], params['b_vn'][None, :])

    gat = params['gat']
    weM = jnp.stack([p['W_e'] @ p['a_e'] for p in gat], axis=1)
    weM = jnp.pad(weM, ((0, 0), (0, 5)))
    CE = _k_ce(edge_attr, weM)

    def apack_of(p, l):
        cv = 0.5 * weM[0, l]
        return jnp.concatenate([
            p['a_src'][None, :], p['a_dst'][None, :],
            jnp.full((1, HID), cv, jnp.float32),
            jnp.zeros((5, HID), jnp.float32)], axis=0)

    def pk_of(p):
        return jnp.concatenate([
            p['b'][None, :], p['ln_g'][None, :], p['ln_b'][None, :],
            jnp.zeros((5, HID), jnp.float32)], axis=0)

    hvs = [hv]
    for l in range(2):
        p = gat[l]
        WxA, SCAL, Wxv = _k_att(h, hv, p['W'], apack_of(p, l),
                                p['a_src'][:, None])
        U2 = _edge_phase(WxA, SCAL, src, dst, CE[l])
        h, hv = _k_post(U2, WxA, SCAL, h, hv, Wxv, pk_of(p))
        hvs.append(hv)

    p = gat[2]
    hv01 = jnp.concatenate([hvs[0], hvs[1], hvs[2],
                            jnp.zeros((1, HID), jnp.float32)], axis=0)
    out = _k_last(
        h, hv, p['W'], apack_of(p, 2), pk_of(p), hv01, agent_features,
        params['W_ih'].T, params['b_ih'][None, :], params['b_hh'][None, :],
        params['W_op'], params['b_op'][None, :],
        jnp.stack([params['ln2_g'], params['ln2_b']], axis=0),
        params['W_rp'], params['b_rp'][None, :],
        params['W_ag'], params['b_ag'][None, :],
        params['W_v'], params['b_v'][None, :],
        params['W_a'], params['b_a'][None, :])
    return out


# speedup vs baseline: 36.6403x; 1.4845x over previous
import functools

import jax
import jax.numpy as jnp
from jax import lax
from jax.experimental import pallas as pl
from jax.experimental.pallas import tpu as pltpu
from jax.experimental.pallas import tpu_sc as plsc

N_NODES = 10000
NPAD = 10240
N_EDGES = 320000
HID = 128
ROW_BLK = 2048
N_BLKS = NPAD // ROW_BLK
UW = 144

_NEG_SLOPE = 0.2


def _leaky(x):
    return jnp.maximum(x, _NEG_SLOPE * x)


def _ln_rows(x, g, b, eps=1e-5):
    mu = jnp.mean(x, axis=-1, keepdims=True)
    var = jnp.mean((x - mu) ** 2, axis=-1, keepdims=True)
    return (x - mu) * jax.lax.rsqrt(var + eps) * g + b


def _pre_body(x_ref, wne_ref, bne_ref, wvn_ref, bvn_ref, h0_ref, vn_ref, acc):
    i = pl.program_id(0)
    h0 = jax.nn.relu(
        jnp.dot(x_ref[...], wne_ref[...], preferred_element_type=jnp.float32)
        + bne_ref[...]
    )
    h0_ref[...] = h0

    @pl.when(i == 0)
    def _():
        acc[...] = jnp.zeros_like(acc)

    ids = i * ROW_BLK + jax.lax.broadcasted_iota(jnp.int32, (ROW_BLK, 1), 0)
    acc[...] += jnp.sum(jnp.where(ids < N_NODES, h0, 0.0), axis=0,
                        keepdims=True)

    @pl.when(i == N_BLKS - 1)
    def _():
        mean = acc[...] / float(N_NODES)
        vn_ref[...] = jnp.tanh(
            jnp.dot(mean, wvn_ref[...], preferred_element_type=jnp.float32)
            + bvn_ref[...]
        )


def _k_pre(x, W_ne, b_ne, W_vn, b_vn):
    return pl.pallas_call(
        _pre_body,
        grid=(N_BLKS,),
        in_specs=[
            pl.BlockSpec((ROW_BLK, HID), lambda i: (i, 0)),
            pl.BlockSpec((HID, HID), lambda i: (0, 0)),
            pl.BlockSpec((1, HID), lambda i: (0, 0)),
            pl.BlockSpec((HID, HID), lambda i: (0, 0)),
            pl.BlockSpec((1, HID), lambda i: (0, 0)),
        ],
        out_specs=[
            pl.BlockSpec((ROW_BLK, HID), lambda i: (i, 0)),
            pl.BlockSpec((1, HID), lambda i: (0, 0)),
        ],
        out_shape=[
            jax.ShapeDtypeStruct((NPAD, HID), jnp.float32),
            jax.ShapeDtypeStruct((1, HID), jnp.float32),
        ],
        scratch_shapes=[pltpu.VMEM((1, HID), jnp.float32)],
    )(x, W_ne, b_ne, W_vn, b_vn)


_CE_BLK = 12800


def _ce_body(ea_ref, wem_ref, ce_ref):
    ea = ea_ref[...]
    wem = wem_ref[...]
    ce = jnp.dot(ea, wem, preferred_element_type=jnp.float32)
    ce_ref[...] = ce.T


def _k_ce(edge_attr, weM):
    return pl.pallas_call(
        _ce_body,
        grid=(N_EDGES // _CE_BLK,),
        in_specs=[
            pl.BlockSpec((_CE_BLK, 3), lambda i: (i, 0)),
            pl.BlockSpec((3, 8), lambda i: (0, 0)),
        ],
        out_specs=pl.BlockSpec((8, _CE_BLK), lambda i: (0, i)),
        out_shape=jax.ShapeDtypeStruct((8, N_EDGES), jnp.float32),
    )(edge_attr, weM)


def _att_body(h_ref, hv_ref, w_ref, apack_ref, apt_ref, wxa_ref, scal_ref,
              wxv_ref):
    i = pl.program_id(0)
    ap = apack_ref[...]
    cv = apack_ref[2, 0]
    wx = jnp.dot(h_ref[...], w_ref[...], preferred_element_type=jnp.float32)
    wxv = jnp.dot(hv_ref[...], w_ref[...], preferred_element_type=jnp.float32)
    asv = jnp.sum(wxv * ap[0:1, :])
    adv = jnp.sum(wxv * ap[1:2, :])
    acol = jnp.dot(wx, apt_ref[...], preferred_element_type=jnp.float32)
    wxa_ref[...] = jnp.concatenate(
        [wx, acol[:, 0:1], jnp.zeros((wx.shape[0], UW - HID - 1),
                                     jnp.float32)], axis=1)
    sc2 = jax.lax.dot_general(ap, wx, (((1,), (1,)), ((), ())),
                              preferred_element_type=jnp.float32)
    asrc = sc2[0:1, :]
    adst = sc2[1:2, :]
    lv = _leaky(asv + adst + cv)
    ids = i * ROW_BLK + jax.lax.broadcasted_iota(jnp.int32, asrc.shape, 1)
    l2v = jnp.where(ids < N_NODES, _leaky(asrc + adv + cv), -jnp.inf)
    z = jnp.zeros((4,) + asrc.shape[1:], jnp.float32)
    scal_ref[...] = jnp.concatenate([asrc, adst, lv, l2v, z], axis=0)

    @pl.when(i == 0)
    def _():
        wxv_ref[...] = wxv


def _k_att(h, hv, W, apack, apackT):
    return pl.pallas_call(
        _att_body,
        grid=(N_BLKS,),
        in_specs=[
            pl.BlockSpec((ROW_BLK, HID), lambda i: (i, 0)),
            pl.BlockSpec((1, HID), lambda i: (0, 0)),
            pl.BlockSpec((HID, HID), lambda i: (0, 0)),
            pl.BlockSpec((8, HID), lambda i: (0, 0)),
            pl.BlockSpec((HID, 1), lambda i: (0, 0)),
        ],
        out_specs=[
            pl.BlockSpec((ROW_BLK, UW), lambda i: (i, 0)),
            pl.BlockSpec((8, ROW_BLK), lambda i: (0, i)),
            pl.BlockSpec((1, HID), lambda i: (0, 0)),
        ],
        out_shape=[
            jax.ShapeDtypeStruct((NPAD, UW), jnp.float32),
            jax.ShapeDtypeStruct((8, NPAD), jnp.float32),
            jax.ShapeDtypeStruct((1, HID), jnp.float32),
        ],
    )(h, hv, W, apack, apackT)


def _post_body(u2_ref, wx_ref, scal_ref, h_ref, hv_ref, wxv_ref, pk_ref,
               hn_ref, hvn_ref, m_sc, s_sc, acc):
    i = pl.program_id(0)
    b = pk_ref[0:1, :]
    ln_g = pk_ref[1:2, :]
    ln_b = pk_ref[2:3, :]
    u = u2_ref[0, :, 0:HID] + u2_ref[1, :, 0:HID]
    sreal = u2_ref[0, :, HID:HID + 1] + u2_ref[1, :, HID:HID + 1]
    agg = (u + wxv_ref[...]) / (sreal + 1.0) + b
    hn = _ln_rows(agg, ln_g, ln_b)
    hn_ref[...] = jax.nn.relu(hn + h_ref[...])

    wx = wx_ref[...][:, 0:HID]
    lb = scal_ref[3:4, :]
    mloc = jnp.max(lb)

    @pl.when(i == 0)
    def _():
        m_sc[0, 0] = -jnp.inf
        s_sc[0, 0] = 0.0
        acc[...] = jnp.zeros_like(acc)

    m_old = m_sc[0, 0]
    m_new = jnp.maximum(m_old, mloc)
    scale = jnp.where(m_old == -jnp.inf, 0.0, jnp.exp(m_old - m_new))
    wts = jnp.exp(lb - m_new)
    acc[...] = acc[...] * scale + jnp.dot(
        wts, wx, preferred_element_type=jnp.float32
    )
    s_sc[0, 0] = s_sc[0, 0] * scale + jnp.sum(wts)
    m_sc[0, 0] = m_new

    @pl.when(i == N_BLKS - 1)
    def _():
        aggv = acc[...] / s_sc[0, 0] + b
        hnv = _ln_rows(aggv, ln_g, ln_b)
        hvn_ref[...] = jax.nn.relu(hnv + hv_ref[...])


def _k_post(U2, Wx, SCAL, h, hv, Wxv, pk):
    return pl.pallas_call(
        _post_body,
        grid=(N_BLKS,),
        in_specs=[
            pl.BlockSpec((2, ROW_BLK, UW), lambda i: (0, i, 0)),
            pl.BlockSpec((ROW_BLK, UW), lambda i: (i, 0)),
            pl.BlockSpec((8, ROW_BLK), lambda i: (0, i)),
            pl.BlockSpec((ROW_BLK, HID), lambda i: (i, 0)),
            pl.BlockSpec((1, HID), lambda i: (0, 0)),
            pl.BlockSpec((1, HID), lambda i: (0, 0)),
            pl.BlockSpec((8, HID), lambda i: (0, 0)),
        ],
        out_specs=[
            pl.BlockSpec((ROW_BLK, HID), lambda i: (i, 0)),
            pl.BlockSpec((1, HID), lambda i: (0, 0)),
        ],
        out_shape=[
            jax.ShapeDtypeStruct((NPAD, HID), jnp.float32),
            jax.ShapeDtypeStruct((1, HID), jnp.float32),
        ],
        scratch_shapes=[
            pltpu.SMEM((1, 1), jnp.float32),
            pltpu.SMEM((1, 1), jnp.float32),
            pltpu.VMEM((1, HID), jnp.float32),
        ],
    )(U2, Wx, SCAL, h, hv, Wxv, pk)


def _last_body(h_ref, hv_ref, w_ref, apack_ref, pk_ref,
               hv01_ref, agent_ref,
               wiht_ref, bih_ref, bhh_ref, wop_ref, bop_ref, ln2_ref,
               wrp_ref, brp_ref, wag_ref, bag_ref, wv_ref, bv_ref,
               wa_ref, ba_ref,
               out_ref, m_sc, s_sc, acc):
    i = pl.program_id(0)
    a_src = apack_ref[0:1, :]
    a_dst = apack_ref[1:2, :]
    cv = apack_ref[2, 0]
    wx = jnp.dot(h_ref[...], w_ref[...], preferred_element_type=jnp.float32)
    wxv = jnp.dot(hv_ref[...], w_ref[...], preferred_element_type=jnp.float32)
    adv = jnp.sum(wxv * a_dst)
    asrc = jax.lax.dot_general(a_src, wx, (((1,), (1,)), ((), ())),
                               preferred_element_type=jnp.float32)
    ids = i * ROW_BLK + jax.lax.broadcasted_iota(jnp.int32, asrc.shape, 1)
    l2v = jnp.where(ids < N_NODES, _leaky(asrc + adv + cv), -jnp.inf)
    mloc = jnp.max(l2v)

    @pl.when(i == 0)
    def _():
        m_sc[0, 0] = -jnp.inf
        s_sc[0, 0] = 0.0
        acc[...] = jnp.zeros_like(acc)

    m_old = m_sc[0, 0]
    m_new = jnp.maximum(m_old, mloc)
    scale = jnp.where(m_old == -jnp.inf, 0.0, jnp.exp(m_old - m_new))
    wts = jnp.exp(l2v - m_new)
    acc[...] = acc[...] * scale + jnp.dot(
        wts, wx, preferred_element_type=jnp.float32
    )
    s_sc[0, 0] = s_sc[0, 0] * scale + jnp.sum(wts)
    m_sc[0, 0] = m_new

    @pl.when(i == N_BLKS - 1)
    def _():
        b = pk_ref[0:1, :]
        ln_g = pk_ref[1:2, :]
        ln_b = pk_ref[2:3, :]
        aggv = acc[...] / s_sc[0, 0] + b
        hnv = _ln_rows(aggv, ln_g, ln_b)
        hv3 = jax.nn.relu(hnv + hv_ref[...])
        jk = jnp.concatenate([hv01_ref[0:1], hv01_ref[1:2], hv01_ref[2:3],
                              hv3], axis=1)
        gi = jnp.dot(jk, wiht_ref[...],
                     preferred_element_type=jnp.float32) + bih_ref[...]
        bhh = bhh_ref[...]
        r = jax.nn.sigmoid(gi[:, 0:HID] + bhh[:, 0:HID])
        z = jax.nn.sigmoid(gi[:, HID:2 * HID] + bhh[:, HID:2 * HID])
        n = jnp.tanh(gi[:, 2 * HID:] + r * bhh[:, 2 * HID:])
        hidden = (1.0 - z) * n
        op = jnp.dot(hidden, wop_ref[...],
                     preferred_element_type=jnp.float32) + bop_ref[...]
        rec = jax.nn.relu(_ln_rows(op, ln2_ref[0:1, :], ln2_ref[1:2, :]))
        rec = jnp.dot(rec, wrp_ref[...],
                      preferred_element_type=jnp.float32) + brp_ref[...]
        ag = jax.nn.relu(
            jnp.dot(agent_ref[...], wag_ref[...],
                    preferred_element_type=jnp.float32) + bag_ref[...])
        comb = jnp.concatenate([rec, ag], axis=1)
        value = jnp.dot(comb, wv_ref[...],
                        preferred_element_type=jnp.float32) + bv_ref[...]
        adv_q = jnp.dot(comb, wa_ref[...],
                        preferred_element_type=jnp.float32) + ba_ref[...]
        out_ref[...] = value + (adv_q - jnp.mean(adv_q, axis=1, keepdims=True))


def _k_last(h, hv, W, apack, pk, hv01, agent, wiht, bih, bhh, wop, bop,
            ln2, wrp, brp, wag, bag, wv, bv, wa, ba):
    full = lambda r, c: pl.BlockSpec((r, c), lambda i: (0, 0))
    return pl.pallas_call(
        _last_body,
        grid=(N_BLKS,),
        in_specs=[
            pl.BlockSpec((ROW_BLK, HID), lambda i: (i, 0)),
            full(1, HID), full(HID, HID), full(8, HID), full(8, HID),
            full(4, HID), full(1, 16),
            full(512, 384), full(1, 384), full(1, 384),
            full(HID, HID), full(1, HID), full(2, HID),
            full(HID, 512), full(1, 512), full(16, HID), full(1, HID),
            full(640, 1), full(1, 1), full(640, 8), full(1, 8),
        ],
        out_specs=pl.BlockSpec((1, 8), lambda i: (0, 0)),
        out_shape=jax.ShapeDtypeStruct((1, 8), jnp.float32),
        scratch_shapes=[
            pltpu.SMEM((1, 1), jnp.float32),
            pltpu.SMEM((1, 1), jnp.float32),
            pltpu.VMEM((1, HID), jnp.float32),
        ],
    )(h, hv, W, apack, pk, hv01, agent, wiht, bih, bhh, wop, bop,
      ln2, wrp, brp, wag, bag, wv, bv, wa, ba)


_NC = 2
_NS = 16
_NW = _NC * _NS
_EPW = N_EDGES // _NW
_EB = 80
_EBLKS = _EPW // _EB
_RPT = NPAD // _NS


def _sc_edge_body(wxa_hbm, adst_hbm, lv_hbm, src_hbm, dst_hbm,
                  ce_hbm, out_hbm, e_v, *rest):
    bufs = []
    for par in range(2):
        o = par * 11
        bufs.append(dict(
            src=rest[o + 0], dst=rest[o + 1], ce=rest[o + 2],
            ta=rest[o + 3], tl=rest[o + 4], dsc=rest[o + 5],
            sr=rest[o + 6], si=rest[o + 7], srm=rest[o + 8],
            st=rest[o + 9], ss=rest[o + 10]))
    u_sh = rest[22]
    cid = lax.axis_index("c")
    sid = lax.axis_index("s")
    wid = sid * _NC + cid
    ebase = wid * _EPW
    rbase = sid * _RPT
    onehot = jnp.where(lax.iota(jnp.int32, 16) == 0, 1.0, 0.0)
    col128 = jnp.full((16,), HID, jnp.int32)
    iota16 = lax.iota(jnp.int32, 16)
    zero16 = jnp.zeros((16,), jnp.float32)

    def issue_idx(g, p):
        base = ebase + g * _EB
        pltpu.async_copy(src_hbm.at[pl.ds(base, _EB)], p['src'], p['si'])
        pltpu.async_copy(dst_hbm.at[pl.ds(base, _EB)], p['dst'], p['si'])
        pltpu.async_copy(ce_hbm.at[pl.ds(base, _EB)], p['ce'], p['si'])

    def wait_idx(p):
        pltpu.make_async_copy(src_hbm.at[pl.ds(0, _EB)], p['src'], p['si']).wait()
        pltpu.make_async_copy(dst_hbm.at[pl.ds(0, _EB)], p['dst'], p['si']).wait()
        pltpu.make_async_copy(ce_hbm.at[pl.ds(0, _EB)], p['ce'], p['si']).wait()

    def issue_rows(p):
        pltpu.async_copy(wxa_hbm.at[p['src']], p['sr'], p['srm'])

    def wait_rows(p):
        pltpu.make_async_copy(wxa_hbm.at[p['src']], p['sr'], p['srm']).wait()

    def issue_tabs(p):
        pltpu.async_copy(adst_hbm.at[p['dst']], p['ta'], p['st'])
        pltpu.async_copy(lv_hbm.at[p['dst']], p['tl'], p['st'])

    def wait_tabs(p):
        pltpu.make_async_copy(adst_hbm.at[p['dst']], p['ta'], p['st']).wait()
        pltpu.make_async_copy(lv_hbm.at[p['dst']], p['tl'], p['st']).wait()

    def issue_scat(p):
        pltpu.async_copy(p['sr'], u_sh.at[p['dsc']], p['ss'], add=True)

    def wait_scat(p):
        pltpu.make_async_copy(p['sr'], u_sh.at[p['dsc']], p['ss']).wait()

    def compute(p):
        for j in range(_EB // 16):
            dsj = pl.ds(16 * j, 16)
            av = plsc.load_gather(p['sr'], [iota16 + (16 * j), col128])
            t = av + p['ta'][dsj] + p['ce'][dsj]
            lg = jnp.maximum(t, _NEG_SLOPE * t)
            e_v[dsj] = jnp.exp(jnp.minimum(lg - p['tl'][dsj], 60.0))
            p['dsc'][dsj] = p['dst'][dsj]

        def srow(j, c2):
            ev = plsc.load_gather(e_v, [jnp.full((16,), j, jnp.int32)])
            for k in range(HID // 16):
                p['sr'][j, pl.ds(16 * k, 16)] = p['sr'][j, pl.ds(16 * k, 16)] * ev
            p['sr'][j, pl.ds(HID, 16)] = ev * onehot
            return c2

        lax.fori_loop(0, _EB, srow, 0)

    issue_idx(0, bufs[0])

    def zrow(j, c):
        for k in range(UW // 16):
            bufs[1]['sr'][j, pl.ds(16 * k, 16)] = zero16
        return c

    lax.fori_loop(0, _EB, zrow, 0)

    def zcp(j, c):
        pltpu.sync_copy(bufs[1]['sr'], u_sh.at[pl.ds(rbase + j * _EB, _EB)])
        return c

    lax.fori_loop(0, _RPT // _EB, zcp, 0)
    plsc.subcore_barrier()
    wait_idx(bufs[0])
    issue_rows(bufs[0])
    issue_tabs(bufs[0])

    def phase(g, p, q, has_next):
        if has_next:
            issue_idx(g + 1, q)
        wait_rows(p)
        wait_tabs(p)
        compute(p)
        issue_scat(p)

        @pl.when(g >= 1)
        def _():
            wait_scat(q)

        if has_next:
            wait_idx(q)
            issue_rows(q)
            issue_tabs(q)

    def two(i, c):
        g = 2 * i
        phase(g, bufs[0], bufs[1], True)
        phase(g + 1, bufs[1], bufs[0], True)
        return c

    lax.fori_loop(0, (_EBLKS - 1) // 2, two, 0)
    phase(_EBLKS - 1, bufs[0], bufs[1], False)
    wait_scat(bufs[0])
    plsc.subcore_barrier()
    pltpu.sync_copy(u_sh.at[pl.ds(rbase, _RPT)],
                    out_hbm.at[cid, pl.ds(rbase, _RPT)])


def _sc_scratch_types():
    per_par = [
        pltpu.VMEM((_EB,), jnp.int32),
        pltpu.VMEM((_EB,), jnp.int32),
        pltpu.VMEM((_EB,), jnp.float32),
        pltpu.VMEM((_EB,), jnp.float32),
        pltpu.VMEM((_EB,), jnp.float32),
        pltpu.VMEM((_EB,), jnp.int32),
        pltpu.VMEM((_EB, UW), jnp.float32),
        pltpu.SemaphoreType.DMA,
        pltpu.SemaphoreType.DMA,
        pltpu.SemaphoreType.DMA,
        pltpu.SemaphoreType.DMA,
    ]
    return ([pltpu.VMEM((_EB,), jnp.float32)] + per_par + per_par
            + [pltpu.VMEM_SHARED((NPAD, UW), jnp.float32)])


@functools.partial(
    pl.kernel,
    mesh=plsc.VectorSubcoreMesh(core_axis_name="c", subcore_axis_name="s"),
    out_type=jax.ShapeDtypeStruct((2, NPAD, UW), jnp.float32),
    compiler_params=pltpu.CompilerParams(needs_layout_passes=False,
                                         use_tc_tiling_on_sc=False),
    scratch_types=_sc_scratch_types(),
)
def _k_sc(wxa_hbm, adst_hbm, lv_hbm, src_hbm, dst_hbm, ce_hbm,
          out_hbm, *rest):
    _sc_edge_body(wxa_hbm, adst_hbm, lv_hbm, src_hbm, dst_hbm,
                  ce_hbm, out_hbm, *rest)


def _edge_phase(WxA, SCAL, src, dst, ce):
    return _k_sc(WxA, SCAL[1], SCAL[2], src, dst, ce)


def kernel(x, edge_index, edge_attr, agent_features, params):
    src = edge_index[0].astype(jnp.int32)
    dst = edge_index[1].astype(jnp.int32)
    xp = jnp.pad(x, ((0, NPAD - N_NODES), (0, 0)))

    h, hv = _k_pre(xp, params['W_ne'], params['b_ne'][None, :],
                   params['W_vn'], params['b_vn'][None, :])

    gat = params['gat']
    weM = jnp.stack([p['W_e'] @ p['a_e'] for p in gat], axis=1)
    weM = jnp.pad(weM, ((0, 0), (0, 5)))
    CE = _k_ce(edge_attr, weM)

    def apack_of(p, l):
        cv = 0.5 * weM[0, l]
        return jnp.concatenate([
            p['a_src'][None, :], p['a_dst'][None, :],
            jnp.full((1, HID), cv, jnp.float32),
            jnp.zeros((5, HID), jnp.float32)], axis=0)

    def pk_of(p):
        return jnp.concatenate([
            p['b'][None, :], p['ln_g'][None, :], p['ln_b'][None, :],
            jnp.zeros((5, HID), jnp.float32)], axis=0)

    hvs = [hv]
    for l in range(2):
        p = gat[l]
        WxA, SCAL, Wxv = _k_att(h, hv, p['W'], apack_of(p, l),
                                p['a_src'][:, None])
        U2 = _edge_phase(WxA, SCAL, src, dst, CE[l])
        h, hv = _k_post(U2, WxA, SCAL, h, hv, Wxv, pk_of(p))
        hvs.append(hv)

    p = gat[2]
    hv01 = jnp.concatenate([hvs[0], hvs[1], hvs[2],
                            jnp.zeros((1, HID), jnp.float32)], axis=0)
    out = _k_last(
        h, hv, p['W'], apack_of(p, 2), pk_of(p), hv01, agent_features,
        params['W_ih'].T, params['b_ih'][None, :], params['b_hh'][None, :],
        params['W_op'], params['b_op'][None, :],
        jnp.stack([params['ln2_g'], params['ln2_b']], axis=0),
        params['W_rp'], params['b_rp'][None, :],
        params['W_ag'], params['b_ag'][None, :],
        params['W_v'], params['b_v'][None, :],
        params['W_a'], params['b_a'][None, :])
    return out


# speedup vs baseline: 50.5471x; 1.3796x over previous
import functools

import jax
import jax.numpy as jnp
from jax import lax
from jax.experimental import pallas as pl
from jax.experimental.pallas import tpu as pltpu
from jax.experimental.pallas import tpu_sc as plsc

N_NODES = 10000
NPAD = 10240
N_EDGES = 320000
HID = 128
ROW_BLK = 2048
N_BLKS = NPAD // ROW_BLK
UW = 144

_NEG_SLOPE = 0.2


def _leaky(x):
    return jnp.maximum(x, _NEG_SLOPE * x)


def _ln_rows(x, g, b, eps=1e-5):
    mu = jnp.mean(x, axis=-1, keepdims=True)
    var = jnp.mean((x - mu) ** 2, axis=-1, keepdims=True)
    return (x - mu) * jax.lax.rsqrt(var + eps) * g + b


def _pre_body(x_ref, wne_ref, bne_ref, wvn_ref, bvn_ref, h0_ref, vn_ref, acc):
    i = pl.program_id(0)
    h0 = jax.nn.relu(
        jnp.dot(x_ref[...], wne_ref[...], preferred_element_type=jnp.float32)
        + bne_ref[...]
    )
    h0_ref[...] = h0

    @pl.when(i == 0)
    def _():
        acc[...] = jnp.zeros_like(acc)

    ids = i * ROW_BLK + jax.lax.broadcasted_iota(jnp.int32, (ROW_BLK, 1), 0)
    acc[...] += jnp.sum(jnp.where(ids < N_NODES, h0, 0.0), axis=0,
                        keepdims=True)

    @pl.when(i == N_BLKS - 1)
    def _():
        mean = acc[...] / float(N_NODES)
        vn_ref[...] = jnp.tanh(
            jnp.dot(mean, wvn_ref[...], preferred_element_type=jnp.float32)
            + bvn_ref[...]
        )


def _k_pre(x, W_ne, b_ne, W_vn, b_vn):
    return pl.pallas_call(
        _pre_body,
        grid=(N_BLKS,),
        in_specs=[
            pl.BlockSpec((ROW_BLK, HID), lambda i: (i, 0)),
            pl.BlockSpec((HID, HID), lambda i: (0, 0)),
            pl.BlockSpec((1, HID), lambda i: (0, 0)),
            pl.BlockSpec((HID, HID), lambda i: (0, 0)),
            pl.BlockSpec((1, HID), lambda i: (0, 0)),
        ],
        out_specs=[
            pl.BlockSpec((ROW_BLK, HID), lambda i: (i, 0)),
            pl.BlockSpec((1, HID), lambda i: (0, 0)),
        ],
        out_shape=[
            jax.ShapeDtypeStruct((NPAD, HID), jnp.float32),
            jax.ShapeDtypeStruct((1, HID), jnp.float32),
        ],
        scratch_shapes=[pltpu.VMEM((1, HID), jnp.float32)],
    )(x, W_ne, b_ne, W_vn, b_vn)


_CE_BLK = 12800


def _ce_body(ea_ref, wem_ref, ce_ref):
    ea = ea_ref[...]
    wem = wem_ref[...]
    ce = jnp.dot(ea, wem, preferred_element_type=jnp.float32)
    ce_ref[...] = ce.T


def _k_ce(edge_attr, weM):
    return pl.pallas_call(
        _ce_body,
        grid=(N_EDGES // _CE_BLK,),
        in_specs=[
            pl.BlockSpec((_CE_BLK, 3), lambda i: (i, 0)),
            pl.BlockSpec((3, 8), lambda i: (0, 0)),
        ],
        out_specs=pl.BlockSpec((8, _CE_BLK), lambda i: (0, i)),
        out_shape=jax.ShapeDtypeStruct((8, N_EDGES), jnp.float32),
    )(edge_attr, weM)


def _att_body(h_ref, hv_ref, w_ref, apack_ref, apt_ref, wxa_ref, scal_ref,
              wxv_ref):
    i = pl.program_id(0)
    ap = apack_ref[...]
    cv = apack_ref[2, 0]
    wx = jnp.dot(h_ref[...], w_ref[...], preferred_element_type=jnp.float32)
    wxv = jnp.dot(hv_ref[...], w_ref[...], preferred_element_type=jnp.float32)
    asv = jnp.sum(wxv * ap[0:1, :])
    adv = jnp.sum(wxv * ap[1:2, :])
    acol = jnp.dot(wx, apt_ref[...], preferred_element_type=jnp.float32)
    wxa_ref[...] = jnp.concatenate(
        [wx, acol[:, 0:1], jnp.zeros((wx.shape[0], UW - HID - 1),
                                     jnp.float32)], axis=1)
    sc2 = jax.lax.dot_general(ap, wx, (((1,), (1,)), ((), ())),
                              preferred_element_type=jnp.float32)
    asrc = sc2[0:1, :]
    adst = sc2[1:2, :]
    lv = _leaky(asv + adst + cv)
    ids = i * ROW_BLK + jax.lax.broadcasted_iota(jnp.int32, asrc.shape, 1)
    l2v = jnp.where(ids < N_NODES, _leaky(asrc + adv + cv), -jnp.inf)
    z = jnp.zeros((4,) + asrc.shape[1:], jnp.float32)
    scal_ref[...] = jnp.concatenate([asrc, adst, lv, l2v, z], axis=0)

    @pl.when(i == 0)
    def _():
        wxv_ref[...] = wxv


def _k_att(h, hv, W, apack, apackT):
    return pl.pallas_call(
        _att_body,
        grid=(N_BLKS,),
        in_specs=[
            pl.BlockSpec((ROW_BLK, HID), lambda i: (i, 0)),
            pl.BlockSpec((1, HID), lambda i: (0, 0)),
            pl.BlockSpec((HID, HID), lambda i: (0, 0)),
            pl.BlockSpec((8, HID), lambda i: (0, 0)),
            pl.BlockSpec((HID, 1), lambda i: (0, 0)),
        ],
        out_specs=[
            pl.BlockSpec((ROW_BLK, UW), lambda i: (i, 0)),
            pl.BlockSpec((8, ROW_BLK), lambda i: (0, i)),
            pl.BlockSpec((1, HID), lambda i: (0, 0)),
        ],
        out_shape=[
            jax.ShapeDtypeStruct((NPAD, UW), jnp.float32),
            jax.ShapeDtypeStruct((8, NPAD), jnp.float32),
            jax.ShapeDtypeStruct((1, HID), jnp.float32),
        ],
    )(h, hv, W, apack, apackT)


def _post_body(u2_ref, wx_ref, scal_ref, h_ref, hv_ref, wxv_ref, pk_ref,
               hn_ref, hvn_ref, m_sc, s_sc, acc):
    i = pl.program_id(0)
    b = pk_ref[0:1, :]
    ln_g = pk_ref[1:2, :]
    ln_b = pk_ref[2:3, :]
    u = u2_ref[0, :, 0:HID] + u2_ref[1, :, 0:HID]
    sreal = u2_ref[0, :, HID:HID + 1] + u2_ref[1, :, HID:HID + 1]
    agg = (u + wxv_ref[...]) / (sreal + 1.0) + b
    hn = _ln_rows(agg, ln_g, ln_b)
    hn_ref[...] = jax.nn.relu(hn + h_ref[...])

    wx = wx_ref[...][:, 0:HID]
    lb = scal_ref[3:4, :]
    mloc = jnp.max(lb)

    @pl.when(i == 0)
    def _():
        m_sc[0, 0] = -jnp.inf
        s_sc[0, 0] = 0.0
        acc[...] = jnp.zeros_like(acc)

    m_old = m_sc[0, 0]
    m_new = jnp.maximum(m_old, mloc)
    scale = jnp.where(m_old == -jnp.inf, 0.0, jnp.exp(m_old - m_new))
    wts = jnp.exp(lb - m_new)
    acc[...] = acc[...] * scale + jnp.dot(
        wts, wx, preferred_element_type=jnp.float32
    )
    s_sc[0, 0] = s_sc[0, 0] * scale + jnp.sum(wts)
    m_sc[0, 0] = m_new

    @pl.when(i == N_BLKS - 1)
    def _():
        aggv = acc[...] / s_sc[0, 0] + b
        hnv = _ln_rows(aggv, ln_g, ln_b)
        hvn_ref[...] = jax.nn.relu(hnv + hv_ref[...])


def _k_post(U2, Wx, SCAL, h, hv, Wxv, pk):
    return pl.pallas_call(
        _post_body,
        grid=(N_BLKS,),
        in_specs=[
            pl.BlockSpec((2, ROW_BLK, UW), lambda i: (0, i, 0)),
            pl.BlockSpec((ROW_BLK, UW), lambda i: (i, 0)),
            pl.BlockSpec((8, ROW_BLK), lambda i: (0, i)),
            pl.BlockSpec((ROW_BLK, HID), lambda i: (i, 0)),
            pl.BlockSpec((1, HID), lambda i: (0, 0)),
            pl.BlockSpec((1, HID), lambda i: (0, 0)),
            pl.BlockSpec((8, HID), lambda i: (0, 0)),
        ],
        out_specs=[
            pl.BlockSpec((ROW_BLK, HID), lambda i: (i, 0)),
            pl.BlockSpec((1, HID), lambda i: (0, 0)),
        ],
        out_shape=[
            jax.ShapeDtypeStruct((NPAD, HID), jnp.float32),
            jax.ShapeDtypeStruct((1, HID), jnp.float32),
        ],
        scratch_shapes=[
            pltpu.SMEM((1, 1), jnp.float32),
            pltpu.SMEM((1, 1), jnp.float32),
            pltpu.VMEM((1, HID), jnp.float32),
        ],
    )(U2, Wx, SCAL, h, hv, Wxv, pk)


def _last_body(h_ref, hv_ref, w_ref, apack_ref, pk_ref,
               hv01_ref, agent_ref,
               wiht_ref, bih_ref, bhh_ref, wop_ref, bop_ref, ln2_ref,
               wrp_ref, brp_ref, wag_ref, bag_ref, wv_ref, bv_ref,
               wa_ref, ba_ref,
               out_ref, m_sc, s_sc, acc):
    i = pl.program_id(0)
    a_src = apack_ref[0:1, :]
    a_dst = apack_ref[1:2, :]
    cv = apack_ref[2, 0]
    wx = jnp.dot(h_ref[...], w_ref[...], preferred_element_type=jnp.float32)
    wxv = jnp.dot(hv_ref[...], w_ref[...], preferred_element_type=jnp.float32)
    adv = jnp.sum(wxv * a_dst)
    asrc = jax.lax.dot_general(a_src, wx, (((1,), (1,)), ((), ())),
                               preferred_element_type=jnp.float32)
    ids = i * ROW_BLK + jax.lax.broadcasted_iota(jnp.int32, asrc.shape, 1)
    l2v = jnp.where(ids < N_NODES, _leaky(asrc + adv + cv), -jnp.inf)
    mloc = jnp.max(l2v)

    @pl.when(i == 0)
    def _():
        m_sc[0, 0] = -jnp.inf
        s_sc[0, 0] = 0.0
        acc[...] = jnp.zeros_like(acc)

    m_old = m_sc[0, 0]
    m_new = jnp.maximum(m_old, mloc)
    scale = jnp.where(m_old == -jnp.inf, 0.0, jnp.exp(m_old - m_new))
    wts = jnp.exp(l2v - m_new)
    acc[...] = acc[...] * scale + jnp.dot(
        wts, wx, preferred_element_type=jnp.float32
    )
    s_sc[0, 0] = s_sc[0, 0] * scale + jnp.sum(wts)
    m_sc[0, 0] = m_new

    @pl.when(i == N_BLKS - 1)
    def _():
        b = pk_ref[0:1, :]
        ln_g = pk_ref[1:2, :]
        ln_b = pk_ref[2:3, :]
        aggv = acc[...] / s_sc[0, 0] + b
        hnv = _ln_rows(aggv, ln_g, ln_b)
        hv3 = jax.nn.relu(hnv + hv_ref[...])
        jk = jnp.concatenate([hv01_ref[0:1], hv01_ref[1:2], hv01_ref[2:3],
                              hv3], axis=1)
        gi = jnp.dot(jk, wiht_ref[...],
                     preferred_element_type=jnp.float32) + bih_ref[...]
        bhh = bhh_ref[...]
        r = jax.nn.sigmoid(gi[:, 0:HID] + bhh[:, 0:HID])
        z = jax.nn.sigmoid(gi[:, HID:2 * HID] + bhh[:, HID:2 * HID])
        n = jnp.tanh(gi[:, 2 * HID:] + r * bhh[:, 2 * HID:])
        hidden = (1.0 - z) * n
        op = jnp.dot(hidden, wop_ref[...],
                     preferred_element_type=jnp.float32) + bop_ref[...]
        rec = jax.nn.relu(_ln_rows(op, ln2_ref[0:1, :], ln2_ref[1:2, :]))
        rec = jnp.dot(rec, wrp_ref[...],
                      preferred_element_type=jnp.float32) + brp_ref[...]
        ag = jax.nn.relu(
            jnp.dot(agent_ref[...], wag_ref[...],
                    preferred_element_type=jnp.float32) + bag_ref[...])
        comb = jnp.concatenate([rec, ag], axis=1)
        value = jnp.dot(comb, wv_ref[...],
                        preferred_element_type=jnp.float32) + bv_ref[...]
        adv_q = jnp.dot(comb, wa_ref[...],
                        preferred_element_type=jnp.float32) + ba_ref[...]
        out_ref[...] = value + (adv_q - jnp.mean(adv_q, axis=1, keepdims=True))


def _k_last(h, hv, W, apack, pk, hv01, agent, wiht, bih, bhh, wop, bop,
            ln2, wrp, brp, wag, bag, wv, bv, wa, ba):
    full = lambda r, c: pl.BlockSpec((r, c), lambda i: (0, 0))
    return pl.pallas_call(
        _last_body,
        grid=(N_BLKS,),
        in_specs=[
            pl.BlockSpec((ROW_BLK, HID), lambda i: (i, 0)),
            full(1, HID), full(HID, HID), full(8, HID), full(8, HID),
            full(4, HID), full(1, 16),
            full(512, 384), full(1, 384), full(1, 384),
            full(HID, HID), full(1, HID), full(2, HID),
            full(HID, 512), full(1, 512), full(16, HID), full(1, HID),
            full(640, 1), full(1, 1), full(640, 8), full(1, 8),
        ],
        out_specs=pl.BlockSpec((1, 8), lambda i: (0, 0)),
        out_shape=jax.ShapeDtypeStruct((1, 8), jnp.float32),
        scratch_shapes=[
            pltpu.SMEM((1, 1), jnp.float32),
            pltpu.SMEM((1, 1), jnp.float32),
            pltpu.VMEM((1, HID), jnp.float32),
        ],
    )(h, hv, W, apack, pk, hv01, agent, wiht, bih, bhh, wop, bop,
      ln2, wrp, brp, wag, bag, wv, bv, wa, ba)


_NC = 2
_NS = 16
_NW = _NC * _NS
_EPW = N_EDGES // _NW
_EB = 80
_EBLKS = _EPW // _EB
_RPT = NPAD // _NS


_NBUF = 3


def _sc_edge_body(wxa_hbm, adst_hbm, lv_hbm, src_hbm, dst_hbm,
                  ce_hbm, out_hbm, e_v, *rest):
    bufs = []
    for par in range(_NBUF):
        o = par * 11
        bufs.append(dict(
            src=rest[o + 0], dst=rest[o + 1], ce=rest[o + 2],
            ta=rest[o + 3], tl=rest[o + 4], dsc=rest[o + 5],
            sr=rest[o + 6], si=rest[o + 7], srm=rest[o + 8],
            st=rest[o + 9], ss=rest[o + 10]))
    u_sh = rest[_NBUF * 11]
    cid = lax.axis_index("c")
    sid = lax.axis_index("s")
    wid = sid * _NC + cid
    ebase = wid * _EPW
    rbase = sid * _RPT
    onehot = jnp.where(lax.iota(jnp.int32, 16) == 0, 1.0, 0.0)
    col128 = jnp.full((16,), HID, jnp.int32)
    iota16 = lax.iota(jnp.int32, 16)
    zero16 = jnp.zeros((16,), jnp.float32)

    def issue_idx(g, p):
        base = ebase + g * _EB
        pltpu.async_copy(src_hbm.at[pl.ds(base, _EB)], p['src'], p['si'])
        pltpu.async_copy(dst_hbm.at[pl.ds(base, _EB)], p['dst'], p['si'])
        pltpu.async_copy(ce_hbm.at[pl.ds(base, _EB)], p['ce'], p['si'])

    def wait_idx(p):
        pltpu.make_async_copy(src_hbm.at[pl.ds(0, _EB)], p['src'], p['si']).wait()
        pltpu.make_async_copy(dst_hbm.at[pl.ds(0, _EB)], p['dst'], p['si']).wait()
        pltpu.make_async_copy(ce_hbm.at[pl.ds(0, _EB)], p['ce'], p['si']).wait()

    def issue_rows(p):
        pltpu.async_copy(wxa_hbm.at[p['src']], p['sr'], p['srm'])

    def wait_rows(p):
        pltpu.make_async_copy(wxa_hbm.at[p['src']], p['sr'], p['srm']).wait()

    def issue_tabs(p):
        pltpu.async_copy(adst_hbm.at[p['dst']], p['ta'], p['st'])
        pltpu.async_copy(lv_hbm.at[p['dst']], p['tl'], p['st'])

    def wait_tabs(p):
        pltpu.make_async_copy(adst_hbm.at[p['dst']], p['ta'], p['st']).wait()
        pltpu.make_async_copy(lv_hbm.at[p['dst']], p['tl'], p['st']).wait()

    def issue_scat(p):
        pltpu.async_copy(p['sr'], u_sh.at[p['dsc']], p['ss'], add=True)

    def wait_scat(p):
        pltpu.make_async_copy(p['sr'], u_sh.at[p['dsc']], p['ss']).wait()

    def compute(p):
        for j in range(_EB // 16):
            dsj = pl.ds(16 * j, 16)
            av = plsc.load_gather(p['sr'], [iota16 + (16 * j), col128])
            t = av + p['ta'][dsj] + p['ce'][dsj]
            lg = jnp.maximum(t, _NEG_SLOPE * t)
            e_v[dsj] = jnp.exp(jnp.minimum(lg - p['tl'][dsj], 60.0))
            p['dsc'][dsj] = p['dst'][dsj]

        def srow(i2, c2):
            for u in range(2):
                j = 2 * i2 + u
                ev = plsc.load_gather(e_v, [jnp.full((16,), j, jnp.int32)])
                for k in range(HID // 16):
                    p['sr'][j, pl.ds(16 * k, 16)] = (
                        p['sr'][j, pl.ds(16 * k, 16)] * ev)
                p['sr'][j, pl.ds(HID, 16)] = ev * onehot
            return c2

        lax.fori_loop(0, _EB // 2, srow, 0)

    issue_idx(0, bufs[0])
    issue_idx(1, bufs[1])

    def zrow(j, c):
        for k in range(UW // 16):
            bufs[2]['sr'][j, pl.ds(16 * k, 16)] = zero16
        return c

    lax.fori_loop(0, _EB, zrow, 0)

    def zcp(j, c):
        pltpu.sync_copy(bufs[2]['sr'], u_sh.at[pl.ds(rbase + j * _EB, _EB)])
        return c

    lax.fori_loop(0, _RPT // _EB, zcp, 0)
    plsc.subcore_barrier()
    wait_idx(bufs[0])
    issue_rows(bufs[0])
    issue_tabs(bufs[0])
    wait_idx(bufs[1])
    issue_rows(bufs[1])
    issue_tabs(bufs[1])

    def phase(g, p, q, r, has_next2):
        if has_next2:
            issue_idx(g + 2, r)
        wait_rows(p)
        wait_tabs(p)
        compute(p)
        issue_scat(p)

        @pl.when(g >= 1)
        def _():
            wait_scat(r)

        if has_next2:
            wait_idx(r)
            issue_rows(r)
            issue_tabs(r)

    def three(i, c):
        g = 3 * i
        phase(g, bufs[0], bufs[1], bufs[2], True)
        phase(g + 1, bufs[1], bufs[2], bufs[0], True)
        phase(g + 2, bufs[2], bufs[0], bufs[1], True)
        return c

    lax.fori_loop(0, (_EBLKS - 2) // 3, three, 0)
    phase(_EBLKS - 2, bufs[0], bufs[1], bufs[2], False)
    phase(_EBLKS - 1, bufs[1], bufs[2], bufs[0], False)
    wait_scat(bufs[1])
    plsc.subcore_barrier()
    pltpu.sync_copy(u_sh.at[pl.ds(rbase, _RPT)],
                    out_hbm.at[cid, pl.ds(rbase, _RPT)])


def _sc_scratch_types():
    per_par = [
        pltpu.VMEM((_EB,), jnp.int32),
        pltpu.VMEM((_EB,), jnp.int32),
        pltpu.VMEM((_EB,), jnp.float32),
        pltpu.VMEM((_EB,), jnp.float32),
        pltpu.VMEM((_EB,), jnp.float32),
        pltpu.VMEM((_EB,), jnp.int32),
        pltpu.VMEM((_EB, UW), jnp.float32),
        pltpu.SemaphoreType.DMA,
        pltpu.SemaphoreType.DMA,
        pltpu.SemaphoreType.DMA,
        pltpu.SemaphoreType.DMA,
    ]
    return ([pltpu.VMEM((_EB,), jnp.float32)] + per_par * _NBUF
            + [pltpu.VMEM_SHARED((NPAD, UW), jnp.float32)])


@functools.partial(
    pl.kernel,
    mesh=plsc.VectorSubcoreMesh(core_axis_name="c", subcore_axis_name="s"),
    out_type=jax.ShapeDtypeStruct((2, NPAD, UW), jnp.float32),
    compiler_params=pltpu.CompilerParams(needs_layout_passes=False,
                                         use_tc_tiling_on_sc=False),
    scratch_types=_sc_scratch_types(),
)
def _k_sc(wxa_hbm, adst_hbm, lv_hbm, src_hbm, dst_hbm, ce_hbm,
          out_hbm, *rest):
    _sc_edge_body(wxa_hbm, adst_hbm, lv_hbm, src_hbm, dst_hbm,
                  ce_hbm, out_hbm, *rest)


def _edge_phase(WxA, SCAL, src, dst, ce):
    return _k_sc(WxA, SCAL[1], SCAL[2], src, dst, ce)


def kernel(x, edge_index, edge_attr, agent_features, params):
    src = edge_index[0].astype(jnp.int32)
    dst = edge_index[1].astype(jnp.int32)
    xp = jnp.pad(x, ((0, NPAD - N_NODES), (0, 0)))

    h, hv = _k_pre(xp, params['W_ne'], params['b_ne'][None, :],
                   params['W_vn'], params['b_vn'][None, :])

    gat = params['gat']
    weM = jnp.stack([p['W_e'] @ p['a_e'] for p in gat], axis=1)
    weM = jnp.pad(weM, ((0, 0), (0, 5)))
    CE = _k_ce(edge_attr, weM)

    def apack_of(p, l):
        cv = 0.5 * weM[0, l]
        return jnp.concatenate([
            p['a_src'][None, :], p['a_dst'][None, :],
            jnp.full((1, HID), cv, jnp.float32),
            jnp.zeros((5, HID), jnp.float32)], axis=0)

    def pk_of(p):
        return jnp.concatenate([
            p['b'][None, :], p['ln_g'][None, :], p['ln_b'][None, :],
            jnp.zeros((5, HID), jnp.float32)], axis=0)

    hvs = [hv]
    for l in range(2):
        p = gat[l]
        WxA, SCAL, Wxv = _k_att(h, hv, p['W'], apack_of(p, l),
                                p['a_src'][:, None])
        U2 = _edge_phase(WxA, SCAL, src, dst, CE[l])
        h, hv = _k_post(U2, WxA, SCAL, h, hv, Wxv, pk_of(p))
        hvs.append(hv)

    p = gat[2]
    hv01 = jnp.concatenate([hvs[0], hvs[1], hvs[2],
                            jnp.zeros((1, HID), jnp.float32)], axis=0)
    out = _k_last(
        h, hv, p['W'], apack_of(p, 2), pk_of(p), hv01, agent_features,
        params['W_ih'].T, params['b_ih'][None, :], params['b_hh'][None, :],
        params['W_op'], params['b_op'][None, :],
        jnp.stack([params['ln2_g'], params['ln2_b']], axis=0),
        params['W_rp'], params['b_rp'][None, :],
        params['W_ag'], params['b_ag'][None, :],
        params['W_v'], params['b_v'][None, :],
        params['W_a'], params['b_a'][None, :])
    return out


# speedup vs baseline: 51.9602x; 1.0280x over previous
import functools

import jax
import jax.numpy as jnp
from jax import lax
from jax.experimental import pallas as pl
from jax.experimental.pallas import tpu as pltpu
from jax.experimental.pallas import tpu_sc as plsc

N_NODES = 10000
NPAD = 10240
N_EDGES = 320000
HID = 128
ROW_BLK = 2048
N_BLKS = NPAD // ROW_BLK
UW = 144

_NEG_SLOPE = 0.2


def _leaky(x):
    return jnp.maximum(x, _NEG_SLOPE * x)


def _ln_rows(x, g, b, eps=1e-5):
    mu = jnp.mean(x, axis=-1, keepdims=True)
    var = jnp.mean((x - mu) ** 2, axis=-1, keepdims=True)
    return (x - mu) * jax.lax.rsqrt(var + eps) * g + b


_CE_BLK = 12800


def _ce_body(ea_ref, wem_ref, ce_ref):
    ea = ea_ref[...]
    wem = wem_ref[...]
    ce = jnp.dot(ea, wem, preferred_element_type=jnp.float32)
    ce_ref[...] = ce.T


def _k_ce(edge_attr, weM):
    return pl.pallas_call(
        _ce_body,
        grid=(N_EDGES // _CE_BLK,),
        in_specs=[
            pl.BlockSpec((_CE_BLK, 3), lambda i: (i, 0)),
            pl.BlockSpec((3, 8), lambda i: (0, 0)),
        ],
        out_specs=pl.BlockSpec((8, _CE_BLK), lambda i: (0, i)),
        out_shape=jax.ShapeDtypeStruct((8, N_EDGES), jnp.float32),
    )(edge_attr, weM)


def _att_block(h1, w_ref, apack_ref, ascol_ref, asum, i):
    wx = jnp.dot(h1, w_ref[...], preferred_element_type=jnp.float32)
    acol = jnp.dot(wx, ascol_ref[...], preferred_element_type=jnp.float32)
    wxa = jnp.concatenate(
        [wx, acol[:, 0:1],
         jnp.zeros((wx.shape[0], UW - HID - 1), jnp.float32)], axis=1)
    sc2 = jax.lax.dot_general(apack_ref[0:2, :], wx, (((1,), (1,)), ((), ())),
                              preferred_element_type=jnp.float32)
    asum[:, pl.ds(i * ROW_BLK, ROW_BLK)] = sc2
    return wxa


def _att_epilogue(hvn, w_ref, apack_ref, asum, scal_ref, wxv_ref):
    wxv = jnp.dot(hvn, w_ref[...], preferred_element_type=jnp.float32)
    wxv_ref[...] = wxv
    ap = apack_ref[...]
    cv = apack_ref[2, 0]
    asv = jnp.sum(wxv * ap[0:1, :])
    adv = jnp.sum(wxv * ap[1:2, :])
    asrc = asum[0:1, :]
    adst = asum[1:2, :]
    lv = _leaky(asv + adst + cv)
    ids = jax.lax.broadcasted_iota(jnp.int32, (1, NPAD), 1)
    l2v = jnp.where(ids < N_NODES, _leaky(asrc + adv + cv), -jnp.inf)
    scal_ref[...] = jnp.concatenate(
        [asrc, adst, lv, l2v, jnp.zeros((4, NPAD), jnp.float32)], axis=0)


def _post_block(u2_ref, h_ref, wxv_ref, pk_ref):
    b = pk_ref[0:1, :]
    g = pk_ref[1:2, :]
    bb = pk_ref[2:3, :]
    u = u2_ref[0, :, 0:HID] + u2_ref[1, :, 0:HID]
    sreal = u2_ref[0, :, HID:HID + 1] + u2_ref[1, :, HID:HID + 1]
    agg = (u + wxv_ref[...]) / (sreal + 1.0) + b
    return jax.nn.relu(_ln_rows(agg, g, bb) + h_ref[...])


def _vnsoft_step(i, wxa_ref, scal_ref, m_sc, s_sc, acc):
    wx = wxa_ref[...][:, 0:HID]
    lb = scal_ref[3:4, :]
    mloc = jnp.max(lb)

    @pl.when(i == 0)
    def _():
        m_sc[0, 0] = -jnp.inf
        s_sc[0, 0] = 0.0
        acc[...] = jnp.zeros_like(acc)

    m_old = m_sc[0, 0]
    m_new = jnp.maximum(m_old, mloc)
    scale = jnp.where(m_old == -jnp.inf, 0.0, jnp.exp(m_old - m_new))
    wts = jnp.exp(lb - m_new)
    acc[...] = acc[...] * scale + jnp.dot(
        wts, wx, preferred_element_type=jnp.float32)
    s_sc[0, 0] = s_sc[0, 0] * scale + jnp.sum(wts)
    m_sc[0, 0] = m_new


def _vnsoft_fin(hv_ref, pk_ref, m_sc, s_sc, acc):
    b = pk_ref[0:1, :]
    g = pk_ref[1:2, :]
    bb = pk_ref[2:3, :]
    aggv = acc[...] / s_sc[0, 0] + b
    return jax.nn.relu(_ln_rows(aggv, g, bb) + hv_ref[...])


def _preatt_body(x_ref, wne_ref, bne_ref, wvn_ref, bvn_ref, w_ref, apack_ref,
                 ascol_ref, h_ref, wxa_ref, scal_ref, hv_ref, wxv_ref,
                 csum, asum):
    i = pl.program_id(0)
    h0 = jax.nn.relu(
        jnp.dot(x_ref[...], wne_ref[...], preferred_element_type=jnp.float32)
        + bne_ref[...])
    h_ref[...] = h0

    @pl.when(i == 0)
    def _():
        csum[...] = jnp.zeros_like(csum)

    ids = i * ROW_BLK + jax.lax.broadcasted_iota(jnp.int32, (ROW_BLK, 1), 0)
    csum[...] += jnp.sum(jnp.where(ids < N_NODES, h0, 0.0), axis=0,
                         keepdims=True)
    wxa_ref[...] = _att_block(h0, w_ref, apack_ref, ascol_ref, asum, i)

    @pl.when(i == N_BLKS - 1)
    def _():
        mean = csum[...] / float(N_NODES)
        vn = jnp.tanh(
            jnp.dot(mean, wvn_ref[...], preferred_element_type=jnp.float32)
            + bvn_ref[...])
        hv_ref[...] = vn
        _att_epilogue(vn, w_ref, apack_ref, asum, scal_ref, wxv_ref)


def _k_preatt(x, W_ne, b_ne, W_vn, b_vn, W, apack, ascol):
    full = lambda r, c: pl.BlockSpec((r, c), lambda i: (0, 0))
    return pl.pallas_call(
        _preatt_body,
        grid=(N_BLKS,),
        in_specs=[
            pl.BlockSpec((ROW_BLK, HID), lambda i: (i, 0)),
            full(HID, HID), full(1, HID), full(HID, HID), full(1, HID),
            full(HID, HID), full(8, HID), full(HID, 1),
        ],
        out_specs=[
            pl.BlockSpec((ROW_BLK, HID), lambda i: (i, 0)),
            pl.BlockSpec((ROW_BLK, UW), lambda i: (i, 0)),
            full(8, NPAD), full(1, HID), full(1, HID),
        ],
        out_shape=[
            jax.ShapeDtypeStruct((NPAD, HID), jnp.float32),
            jax.ShapeDtypeStruct((NPAD, UW), jnp.float32),
            jax.ShapeDtypeStruct((8, NPAD), jnp.float32),
            jax.ShapeDtypeStruct((1, HID), jnp.float32),
            jax.ShapeDtypeStruct((1, HID), jnp.float32),
        ],
        scratch_shapes=[
            pltpu.VMEM((1, HID), jnp.float32),
            pltpu.VMEM((2, NPAD), jnp.float32),
        ],
    )(x, W_ne, b_ne, W_vn, b_vn, W, apack, ascol)


def _fuse_body(u2_ref, wxa_ref, scal_ref, h_ref, hv_ref, wxv_ref, pk_ref,
               w2_ref, apack2_ref, ascol2_ref,
               h2_ref, wxa2_ref, scal2_ref, hv2_ref, wxv2_ref,
               m_sc, s_sc, acc, asum):
    i = pl.program_id(0)
    h1 = _post_block(u2_ref, h_ref, wxv_ref, pk_ref)
    h2_ref[...] = h1
    _vnsoft_step(i, wxa_ref, scal_ref, m_sc, s_sc, acc)
    wxa2_ref[...] = _att_block(h1, w2_ref, apack2_ref, ascol2_ref, asum, i)

    @pl.when(i == N_BLKS - 1)
    def _():
        hvn = _vnsoft_fin(hv_ref, pk_ref, m_sc, s_sc, acc)
        hv2_ref[...] = hvn
        _att_epilogue(hvn, w2_ref, apack2_ref, asum, scal2_ref, wxv2_ref)


def _k_fuse(U2, WxA, SCAL, h, hv, Wxv, pk, W2, apack2, ascol2):
    full = lambda r, c: pl.BlockSpec((r, c), lambda i: (0, 0))
    return pl.pallas_call(
        _fuse_body,
        grid=(N_BLKS,),
        in_specs=[
            pl.BlockSpec((2, ROW_BLK, UW), lambda i: (0, i, 0)),
            pl.BlockSpec((ROW_BLK, UW), lambda i: (i, 0)),
            pl.BlockSpec((8, ROW_BLK), lambda i: (0, i)),
            pl.BlockSpec((ROW_BLK, HID), lambda i: (i, 0)),
            full(1, HID), full(1, HID), full(8, HID),
            full(HID, HID), full(8, HID), full(HID, 1),
        ],
        out_specs=[
            pl.BlockSpec((ROW_BLK, HID), lambda i: (i, 0)),
            pl.BlockSpec((ROW_BLK, UW), lambda i: (i, 0)),
            full(8, NPAD), full(1, HID), full(1, HID),
        ],
        out_shape=[
            jax.ShapeDtypeStruct((NPAD, HID), jnp.float32),
            jax.ShapeDtypeStruct((NPAD, UW), jnp.float32),
            jax.ShapeDtypeStruct((8, NPAD), jnp.float32),
            jax.ShapeDtypeStruct((1, HID), jnp.float32),
            jax.ShapeDtypeStruct((1, HID), jnp.float32),
        ],
        scratch_shapes=[
            pltpu.SMEM((1, 1), jnp.float32),
            pltpu.SMEM((1, 1), jnp.float32),
            pltpu.VMEM((1, HID), jnp.float32),
            pltpu.VMEM((2, NPAD), jnp.float32),
        ],
    )(U2, WxA, SCAL, h, hv, Wxv, pk, W2, apack2, ascol2)


def _plast_body(u2_ref, wxa_ref, scal_ref, h_ref, hv_ref, wxv_ref, pk_ref,
                w3_ref, apack3_ref, pk3_ref, hv0_ref, agent_ref,
                wiht_ref, bih_ref, bhh_ref, wop_ref, bop_ref, ln2_ref,
                wrp_ref, brp_ref, wag_ref, bag_ref, wv_ref, bv_ref,
                wa_ref, ba_ref,
                out_ref, m_sc, s_sc, acc, wx3_s, asum3):
    i = pl.program_id(0)
    h2 = _post_block(u2_ref, h_ref, wxv_ref, pk_ref)
    _vnsoft_step(i, wxa_ref, scal_ref, m_sc, s_sc, acc)
    wx3 = jnp.dot(h2, w3_ref[...], preferred_element_type=jnp.float32)
    wx3_s[pl.ds(i * ROW_BLK, ROW_BLK), :] = wx3
    asum3[:, pl.ds(i * ROW_BLK, ROW_BLK)] = jax.lax.dot_general(
        apack3_ref[0:1, :], wx3, (((1,), (1,)), ((), ())),
        preferred_element_type=jnp.float32)

    @pl.when(i == N_BLKS - 1)
    def _():
        hv2 = _vnsoft_fin(hv_ref, pk_ref, m_sc, s_sc, acc)
        wxv3 = jnp.dot(hv2, w3_ref[...], preferred_element_type=jnp.float32)
        adv3 = jnp.sum(wxv3 * apack3_ref[1:2, :])
        cv3 = apack3_ref[2, 0]
        ids = jax.lax.broadcasted_iota(jnp.int32, (1, NPAD), 1)
        l2v3 = jnp.where(ids < N_NODES,
                         _leaky(asum3[0:1, :] + adv3 + cv3), -jnp.inf)
        m = jnp.max(l2v3)
        wts = jnp.exp(l2v3 - m)
        sv = jnp.sum(wts)
        aggv = jnp.dot(wts, wx3_s[...],
                       preferred_element_type=jnp.float32) / sv + pk3_ref[0:1, :]
        hv3 = jax.nn.relu(
            _ln_rows(aggv, pk3_ref[1:2, :], pk3_ref[2:3, :]) + hv2)
        jk = jnp.concatenate([hv0_ref[...], hv_ref[...], hv2, hv3], axis=1)
        gi = jnp.dot(jk, wiht_ref[...],
                     preferred_element_type=jnp.float32) + bih_ref[...]
        bhh = bhh_ref[...]
        r = jax.nn.sigmoid(gi[:, 0:HID] + bhh[:, 0:HID])
        z = jax.nn.sigmoid(gi[:, HID:2 * HID] + bhh[:, HID:2 * HID])
        n = jnp.tanh(gi[:, 2 * HID:] + r * bhh[:, 2 * HID:])
        hidden = (1.0 - z) * n
        op = jnp.dot(hidden, wop_ref[...],
                     preferred_element_type=jnp.float32) + bop_ref[...]
        rec = jax.nn.relu(_ln_rows(op, ln2_ref[0:1, :], ln2_ref[1:2, :]))
        rec = jnp.dot(rec, wrp_ref[...],
                      preferred_element_type=jnp.float32) + brp_ref[...]
        ag = jax.nn.relu(
            jnp.dot(agent_ref[...], wag_ref[...],
                    preferred_element_type=jnp.float32) + bag_ref[...])
        comb = jnp.concatenate([rec, ag], axis=1)
        value = jnp.dot(comb, wv_ref[...],
                        preferred_element_type=jnp.float32) + bv_ref[...]
        adv_q = jnp.dot(comb, wa_ref[...],
                        preferred_element_type=jnp.float32) + ba_ref[...]
        out_ref[...] = value + (adv_q - jnp.mean(adv_q, axis=1, keepdims=True))


def _k_plast(U2, WxA, SCAL, h, hv, Wxv, pk, W3, apack3, pk3, hv0, agent,
             wiht, bih, bhh, wop, bop, ln2, wrp, brp, wag, bag, wv, bv,
             wa, ba):
    full = lambda r, c: pl.BlockSpec((r, c), lambda i: (0, 0))
    return pl.pallas_call(
        _plast_body,
        grid=(N_BLKS,),
        in_specs=[
            pl.BlockSpec((2, ROW_BLK, UW), lambda i: (0, i, 0)),
            pl.BlockSpec((ROW_BLK, UW), lambda i: (i, 0)),
            pl.BlockSpec((8, ROW_BLK), lambda i: (0, i)),
            pl.BlockSpec((ROW_BLK, HID), lambda i: (i, 0)),
            full(1, HID), full(1, HID), full(8, HID),
            full(HID, HID), full(8, HID), full(8, HID),
            full(1, HID), full(1, 16),
            full(512, 384), full(1, 384), full(1, 384),
            full(HID, HID), full(1, HID), full(2, HID),
            full(HID, 512), full(1, 512), full(16, HID), full(1, HID),
            full(640, 1), full(1, 1), full(640, 8), full(1, 8),
        ],
        out_specs=pl.BlockSpec((1, 8), lambda i: (0, 0)),
        out_shape=jax.ShapeDtypeStruct((1, 8), jnp.float32),
        scratch_shapes=[
            pltpu.SMEM((1, 1), jnp.float32),
            pltpu.SMEM((1, 1), jnp.float32),
            pltpu.VMEM((1, HID), jnp.float32),
            pltpu.VMEM((NPAD, HID), jnp.float32),
            pltpu.VMEM((1, NPAD), jnp.float32),
        ],
    )(U2, WxA, SCAL, h, hv, Wxv, pk, W3, apack3, pk3, hv0, agent,
      wiht, bih, bhh, wop, bop, ln2, wrp, brp, wag, bag, wv, bv, wa, ba)


_NC = 2
_NS = 16
_NW = _NC * _NS
_EPW = N_EDGES // _NW
_EB = 80
_EBLKS = _EPW // _EB
_RPT = NPAD // _NS


_NBUF = 3


def _sc_edge_body(wxa_hbm, adst_hbm, lv_hbm, src_hbm, dst_hbm,
                  ce_hbm, out_hbm, e_v, *rest):
    bufs = []
    for par in range(_NBUF):
        o = par * 11
        bufs.append(dict(
            src=rest[o + 0], dst=rest[o + 1], ce=rest[o + 2],
            ta=rest[o + 3], tl=rest[o + 4], dsc=rest[o + 5],
            sr=rest[o + 6], si=rest[o + 7], srm=rest[o + 8],
            st=rest[o + 9], ss=rest[o + 10]))
    u_sh = rest[_NBUF * 11]
    cid = lax.axis_index("c")
    sid = lax.axis_index("s")
    wid = sid * _NC + cid
    ebase = wid * _EPW
    rbase = sid * _RPT
    onehot = jnp.where(lax.iota(jnp.int32, 16) == 0, 1.0, 0.0)
    col128 = jnp.full((16,), HID, jnp.int32)
    iota16 = lax.iota(jnp.int32, 16)
    zero16 = jnp.zeros((16,), jnp.float32)

    def issue_idx(g, p):
        base = ebase + g * _EB
        pltpu.async_copy(src_hbm.at[pl.ds(base, _EB)], p['src'], p['si'])
        pltpu.async_copy(dst_hbm.at[pl.ds(base, _EB)], p['dst'], p['si'])
        pltpu.async_copy(ce_hbm.at[pl.ds(base, _EB)], p['ce'], p['si'])

    def wait_idx(p):
        pltpu.make_async_copy(src_hbm.at[pl.ds(0, _EB)], p['src'], p['si']).wait()
        pltpu.make_async_copy(dst_hbm.at[pl.ds(0, _EB)], p['dst'], p['si']).wait()
        pltpu.make_async_copy(ce_hbm.at[pl.ds(0, _EB)], p['ce'], p['si']).wait()

    def issue_rows(p):
        pltpu.async_copy(wxa_hbm.at[p['src']], p['sr'], p['srm'])

    def wait_rows(p):
        pltpu.make_async_copy(wxa_hbm.at[p['src']], p['sr'], p['srm']).wait()

    def issue_tabs(p):
        pltpu.async_copy(adst_hbm.at[p['dst']], p['ta'], p['st'])
        pltpu.async_copy(lv_hbm.at[p['dst']], p['tl'], p['st'])

    def wait_tabs(p):
        pltpu.make_async_copy(adst_hbm.at[p['dst']], p['ta'], p['st']).wait()
        pltpu.make_async_copy(lv_hbm.at[p['dst']], p['tl'], p['st']).wait()

    def issue_scat(p):
        pltpu.async_copy(p['sr'], u_sh.at[p['dsc']], p['ss'], add=True)

    def wait_scat(p):
        pltpu.make_async_copy(p['sr'], u_sh.at[p['dsc']], p['ss']).wait()

    def compute(p):
        for j in range(_EB // 16):
            dsj = pl.ds(16 * j, 16)
            av = plsc.load_gather(p['sr'], [iota16 + (16 * j), col128])
            t = av + p['ta'][dsj] + p['ce'][dsj]
            lg = jnp.maximum(t, _NEG_SLOPE * t)
            e_v[dsj] = jnp.exp(jnp.minimum(lg - p['tl'][dsj], 60.0))
            p['dsc'][dsj] = p['dst'][dsj]

        def srow(i2, c2):
            for u in range(2):
                j = 2 * i2 + u
                ev = plsc.load_gather(e_v, [jnp.full((16,), j, jnp.int32)])
                for k in range(HID // 16):
                    p['sr'][j, pl.ds(16 * k, 16)] = (
                        p['sr'][j, pl.ds(16 * k, 16)] * ev)
                p['sr'][j, pl.ds(HID, 16)] = ev * onehot
            return c2

        lax.fori_loop(0, _EB // 2, srow, 0)

    issue_idx(0, bufs[0])
    issue_idx(1, bufs[1])

    def zrow(j, c):
        for k in range(UW // 16):
            bufs[2]['sr'][j, pl.ds(16 * k, 16)] = zero16
        return c

    lax.fori_loop(0, _EB, zrow, 0)

    def zcp(j, c):
        pltpu.sync_copy(bufs[2]['sr'], u_sh.at[pl.ds(rbase + j * _EB, _EB)])
        return c

    lax.fori_loop(0, _RPT // _EB, zcp, 0)
    plsc.subcore_barrier()
    wait_idx(bufs[0])
    issue_rows(bufs[0])
    issue_tabs(bufs[0])
    wait_idx(bufs[1])
    issue_rows(bufs[1])
    issue_tabs(bufs[1])

    def phase(g, p, q, r, has_next2):
        if has_next2:
            issue_idx(g + 2, r)
        wait_rows(p)
        wait_tabs(p)
        compute(p)
        issue_scat(p)

        @pl.when(g >= 1)
        def _():
            wait_scat(r)

        if has_next2:
            wait_idx(r)
            issue_rows(r)
            issue_tabs(r)

    def three(i, c):
        g = 3 * i
        phase(g, bufs[0], bufs[1], bufs[2], True)
        phase(g + 1, bufs[1], bufs[2], bufs[0], True)
        phase(g + 2, bufs[2], bufs[0], bufs[1], True)
        return c

    lax.fori_loop(0, (_EBLKS - 2) // 3, three, 0)
    phase(_EBLKS - 2, bufs[0], bufs[1], bufs[2], False)
    phase(_EBLKS - 1, bufs[1], bufs[2], bufs[0], False)
    wait_scat(bufs[1])
    plsc.subcore_barrier()
    pltpu.sync_copy(u_sh.at[pl.ds(rbase, _RPT)],
                    out_hbm.at[cid, pl.ds(rbase, _RPT)])


def _sc_scratch_types():
    per_par = [
        pltpu.VMEM((_EB,), jnp.int32),
        pltpu.VMEM((_EB,), jnp.int32),
        pltpu.VMEM((_EB,), jnp.float32),
        pltpu.VMEM((_EB,), jnp.float32),
        pltpu.VMEM((_EB,), jnp.float32),
        pltpu.VMEM((_EB,), jnp.int32),
        pltpu.VMEM((_EB, UW), jnp.float32),
        pltpu.SemaphoreType.DMA,
        pltpu.SemaphoreType.DMA,
        pltpu.SemaphoreType.DMA,
        pltpu.SemaphoreType.DMA,
    ]
    return ([pltpu.VMEM((_EB,), jnp.float32)] + per_par * _NBUF
            + [pltpu.VMEM_SHARED((NPAD, UW), jnp.float32)])


@functools.partial(
    pl.kernel,
    mesh=plsc.VectorSubcoreMesh(core_axis_name="c", subcore_axis_name="s"),
    out_type=jax.ShapeDtypeStruct((2, NPAD, UW), jnp.float32),
    compiler_params=pltpu.CompilerParams(needs_layout_passes=False,
                                         use_tc_tiling_on_sc=False),
    scratch_types=_sc_scratch_types(),
)
def _k_sc(wxa_hbm, adst_hbm, lv_hbm, src_hbm, dst_hbm, ce_hbm,
          out_hbm, *rest):
    _sc_edge_body(wxa_hbm, adst_hbm, lv_hbm, src_hbm, dst_hbm,
                  ce_hbm, out_hbm, *rest)


def _edge_phase(WxA, SCAL, src, dst, ce):
    return _k_sc(WxA, SCAL[1], SCAL[2], src, dst, ce)


def kernel(x, edge_index, edge_attr, agent_features, params):
    src = edge_index[0].astype(jnp.int32)
    dst = edge_index[1].astype(jnp.int32)
    xp = jnp.pad(x, ((0, NPAD - N_NODES), (0, 0)))

    gat = params['gat']
    weM = jnp.stack([p['W_e'] @ p['a_e'] for p in gat], axis=1)
    weM = jnp.pad(weM, ((0, 0), (0, 5)))
    CE = _k_ce(edge_attr, weM)

    def apack_of(p, l):
        cv = 0.5 * weM[0, l]
        return jnp.concatenate([
            p['a_src'][None, :], p['a_dst'][None, :],
            jnp.full((1, HID), cv, jnp.float32),
            jnp.zeros((5, HID), jnp.float32)], axis=0)

    def pk_of(p):
        return jnp.concatenate([
            p['b'][None, :], p['ln_g'][None, :], p['ln_b'][None, :],
            jnp.zeros((5, HID), jnp.float32)], axis=0)

    p0, p1, p2 = gat[0], gat[1], gat[2]
    h, WxA0, SCAL0, hv0, Wxv0 = _k_preatt(
        xp, params['W_ne'], params['b_ne'][None, :],
        params['W_vn'], params['b_vn'][None, :],
        p0['W'], apack_of(p0, 0), p0['a_src'][:, None])
    U2 = _edge_phase(WxA0, SCAL0, src, dst, CE[0])
    h1, WxA1, SCAL1, hv1, Wxv1 = _k_fuse(
        U2, WxA0, SCAL0, h, hv0, Wxv0, pk_of(p0),
        p1['W'], apack_of(p1, 1), p1['a_src'][:, None])
    U2b = _edge_phase(WxA1, SCAL1, src, dst, CE[1])
    out = _k_plast(
        U2b, WxA1, SCAL1, h1, hv1, Wxv1, pk_of(p1),
        p2['W'], apack_of(p2, 2), pk_of(p2), hv0, agent_features,
        params['W_ih'].T, params['b_ih'][None, :], params['b_hh'][None, :],
        params['W_op'], params['b_op'][None, :],
        jnp.stack([params['ln2_g'], params['ln2_b']], axis=0),
        params['W_rp'], params['b_rp'][None, :],
        params['W_ag'], params['b_ag'][None, :],
        params['W_v'], params['b_v'][None, :],
        params['W_a'], params['b_a'][None, :])
    return out


# speedup vs baseline: 66.5843x; 1.2814x over previous
import functools

import jax
import jax.numpy as jnp
from jax import lax
from jax.experimental import pallas as pl
from jax.experimental.pallas import tpu as pltpu
from jax.experimental.pallas import tpu_sc as plsc

N_NODES = 10000
NPAD = 10240
N_EDGES = 320000
HID = 128
ROW_BLK = 2048
N_BLKS = NPAD // ROW_BLK
UW = 144

_NEG_SLOPE = 0.2


def _leaky(x):
    return jnp.maximum(x, _NEG_SLOPE * x)


def _ln_rows(x, g, b, eps=1e-5):
    mu = jnp.mean(x, axis=-1, keepdims=True)
    var = jnp.mean((x - mu) ** 2, axis=-1, keepdims=True)
    return (x - mu) * jax.lax.rsqrt(var + eps) * g + b


def _att_block(h1, w_ref, apack_ref, ascol_ref, asum, i):
    wx = jnp.dot(h1, w_ref[...], preferred_element_type=jnp.float32)
    acol = jnp.dot(wx, ascol_ref[...], preferred_element_type=jnp.float32)
    wxa = jnp.concatenate(
        [wx, acol[:, 0:1],
         jnp.zeros((wx.shape[0], UW - HID - 1), jnp.float32)], axis=1)
    sc2 = jax.lax.dot_general(apack_ref[0:2, :], wx, (((1,), (1,)), ((), ())),
                              preferred_element_type=jnp.float32)
    asum[:, pl.ds(i * ROW_BLK, ROW_BLK)] = sc2
    return wxa


def _att_epilogue(hvn, w_ref, apack_ref, asum, scal_ref, wxv_ref):
    wxv = jnp.dot(hvn, w_ref[...], preferred_element_type=jnp.float32)
    wxv_ref[...] = wxv
    ap = apack_ref[...]
    cv = apack_ref[2, 0]
    asv = jnp.sum(wxv * ap[0:1, :])
    adv = jnp.sum(wxv * ap[1:2, :])
    asrc = asum[0:1, :]
    adst = asum[1:2, :]
    lv = _leaky(asv + adst + cv)
    ids = jax.lax.broadcasted_iota(jnp.int32, (1, NPAD), 1)
    l2v = jnp.where(ids < N_NODES, _leaky(asrc + adv + cv), -jnp.inf)
    scal_ref[...] = jnp.concatenate(
        [asrc, adst, lv, l2v, jnp.zeros((4, NPAD), jnp.float32)], axis=0)


def _post_block(u2_ref, h_ref, wxv_ref, pk_ref):
    b = pk_ref[0:1, :]
    g = pk_ref[1:2, :]
    bb = pk_ref[2:3, :]
    u = u2_ref[0, :, 0:HID] + u2_ref[1, :, 0:HID]
    sreal = u2_ref[0, :, HID:HID + 1] + u2_ref[1, :, HID:HID + 1]
    agg = (u + wxv_ref[...]) / (sreal + 1.0) + b
    return jax.nn.relu(_ln_rows(agg, g, bb) + h_ref[...])


def _vnsoft_step(i, wxa_ref, scal_ref, m_sc, s_sc, acc):
    wx = wxa_ref[...][:, 0:HID]
    lb = scal_ref[3:4, :]
    mloc = jnp.max(lb)

    @pl.when(i == 0)
    def _():
        m_sc[0, 0] = -jnp.inf
        s_sc[0, 0] = 0.0
        acc[...] = jnp.zeros_like(acc)

    m_old = m_sc[0, 0]
    m_new = jnp.maximum(m_old, mloc)
    scale = jnp.where(m_old == -jnp.inf, 0.0, jnp.exp(m_old - m_new))
    wts = jnp.exp(lb - m_new)
    acc[...] = acc[...] * scale + jnp.dot(
        wts, wx, preferred_element_type=jnp.float32)
    s_sc[0, 0] = s_sc[0, 0] * scale + jnp.sum(wts)
    m_sc[0, 0] = m_new


def _vnsoft_fin(hv_ref, pk_ref, m_sc, s_sc, acc):
    b = pk_ref[0:1, :]
    g = pk_ref[1:2, :]
    bb = pk_ref[2:3, :]
    aggv = acc[...] / s_sc[0, 0] + b
    return jax.nn.relu(_ln_rows(aggv, g, bb) + hv_ref[...])


def _preatt_body(x_ref, wne_ref, bne_ref, wvn_ref, bvn_ref, w_ref, apack_ref,
                 ascol_ref, h_ref, wxa_ref, scal_ref, hv_ref, wxv_ref,
                 csum, asum):
    i = pl.program_id(0)
    h0 = jax.nn.relu(
        jnp.dot(x_ref[...], wne_ref[...], preferred_element_type=jnp.float32)
        + bne_ref[...])
    h_ref[...] = h0

    @pl.when(i == 0)
    def _():
        csum[...] = jnp.zeros_like(csum)

    ids = i * ROW_BLK + jax.lax.broadcasted_iota(jnp.int32, (ROW_BLK, 1), 0)
    csum[...] += jnp.sum(jnp.where(ids < N_NODES, h0, 0.0), axis=0,
                         keepdims=True)
    wxa_ref[...] = _att_block(h0, w_ref, apack_ref, ascol_ref, asum, i)

    @pl.when(i == N_BLKS - 1)
    def _():
        mean = csum[...] / float(N_NODES)
        vn = jnp.tanh(
            jnp.dot(mean, wvn_ref[...], preferred_element_type=jnp.float32)
            + bvn_ref[...])
        hv_ref[...] = vn
        _att_epilogue(vn, w_ref, apack_ref, asum, scal_ref, wxv_ref)


def _k_preatt(x, W_ne, b_ne, W_vn, b_vn, W, apack, ascol):
    full = lambda r, c: pl.BlockSpec((r, c), lambda i: (0, 0))
    return pl.pallas_call(
        _preatt_body,
        grid=(N_BLKS,),
        in_specs=[
            pl.BlockSpec((ROW_BLK, HID), lambda i: (i, 0)),
            full(HID, HID), full(1, HID), full(HID, HID), full(1, HID),
            full(HID, HID), full(8, HID), full(HID, 1),
        ],
        out_specs=[
            pl.BlockSpec((ROW_BLK, HID), lambda i: (i, 0)),
            pl.BlockSpec((ROW_BLK, UW), lambda i: (i, 0)),
            full(8, NPAD), full(1, HID), full(1, HID),
        ],
        out_shape=[
            jax.ShapeDtypeStruct((NPAD, HID), jnp.float32),
            jax.ShapeDtypeStruct((NPAD, UW), jnp.float32),
            jax.ShapeDtypeStruct((8, NPAD), jnp.float32),
            jax.ShapeDtypeStruct((1, HID), jnp.float32),
            jax.ShapeDtypeStruct((1, HID), jnp.float32),
        ],
        scratch_shapes=[
            pltpu.VMEM((1, HID), jnp.float32),
            pltpu.VMEM((2, NPAD), jnp.float32),
        ],
    )(x, W_ne, b_ne, W_vn, b_vn, W, apack, ascol)


def _fuse_body(u2_ref, wxa_ref, scal_ref, h_ref, hv_ref, wxv_ref, pk_ref,
               w2_ref, apack2_ref, ascol2_ref,
               h2_ref, wxa2_ref, scal2_ref, hv2_ref, wxv2_ref,
               m_sc, s_sc, acc, asum):
    i = pl.program_id(0)
    h1 = _post_block(u2_ref, h_ref, wxv_ref, pk_ref)
    h2_ref[...] = h1
    _vnsoft_step(i, wxa_ref, scal_ref, m_sc, s_sc, acc)
    wxa2_ref[...] = _att_block(h1, w2_ref, apack2_ref, ascol2_ref, asum, i)

    @pl.when(i == N_BLKS - 1)
    def _():
        hvn = _vnsoft_fin(hv_ref, pk_ref, m_sc, s_sc, acc)
        hv2_ref[...] = hvn
        _att_epilogue(hvn, w2_ref, apack2_ref, asum, scal2_ref, wxv2_ref)


def _k_fuse(U2, WxA, SCAL, h, hv, Wxv, pk, W2, apack2, ascol2):
    full = lambda r, c: pl.BlockSpec((r, c), lambda i: (0, 0))
    return pl.pallas_call(
        _fuse_body,
        grid=(N_BLKS,),
        in_specs=[
            pl.BlockSpec((2, ROW_BLK, UW), lambda i: (0, i, 0)),
            pl.BlockSpec((ROW_BLK, UW), lambda i: (i, 0)),
            pl.BlockSpec((8, ROW_BLK), lambda i: (0, i)),
            pl.BlockSpec((ROW_BLK, HID), lambda i: (i, 0)),
            full(1, HID), full(1, HID), full(8, HID),
            full(HID, HID), full(8, HID), full(HID, 1),
        ],
        out_specs=[
            pl.BlockSpec((ROW_BLK, HID), lambda i: (i, 0)),
            pl.BlockSpec((ROW_BLK, UW), lambda i: (i, 0)),
            full(8, NPAD), full(1, HID), full(1, HID),
        ],
        out_shape=[
            jax.ShapeDtypeStruct((NPAD, HID), jnp.float32),
            jax.ShapeDtypeStruct((NPAD, UW), jnp.float32),
            jax.ShapeDtypeStruct((8, NPAD), jnp.float32),
            jax.ShapeDtypeStruct((1, HID), jnp.float32),
            jax.ShapeDtypeStruct((1, HID), jnp.float32),
        ],
        scratch_shapes=[
            pltpu.SMEM((1, 1), jnp.float32),
            pltpu.SMEM((1, 1), jnp.float32),
            pltpu.VMEM((1, HID), jnp.float32),
            pltpu.VMEM((2, NPAD), jnp.float32),
        ],
    )(U2, WxA, SCAL, h, hv, Wxv, pk, W2, apack2, ascol2)


def _plast_body(u2_ref, wxa_ref, scal_ref, h_ref, hv_ref, wxv_ref, pk_ref,
                w3_ref, apack3_ref, pk3_ref, hv0_ref, agent_ref,
                wiht_ref, bih_ref, bhh_ref, wop_ref, bop_ref, ln2_ref,
                wrp_ref, brp_ref, wag_ref, bag_ref, wv_ref, bv_ref,
                wa_ref, ba_ref,
                out_ref, m_sc, s_sc, acc, wx3_s, asum3):
    i = pl.program_id(0)
    h2 = _post_block(u2_ref, h_ref, wxv_ref, pk_ref)
    _vnsoft_step(i, wxa_ref, scal_ref, m_sc, s_sc, acc)
    wx3 = jnp.dot(h2, w3_ref[...], preferred_element_type=jnp.float32)
    wx3_s[pl.ds(i * ROW_BLK, ROW_BLK), :] = wx3
    asum3[:, pl.ds(i * ROW_BLK, ROW_BLK)] = jax.lax.dot_general(
        apack3_ref[0:1, :], wx3, (((1,), (1,)), ((), ())),
        preferred_element_type=jnp.float32)

    @pl.when(i == N_BLKS - 1)
    def _():
        hv2 = _vnsoft_fin(hv_ref, pk_ref, m_sc, s_sc, acc)
        wxv3 = jnp.dot(hv2, w3_ref[...], preferred_element_type=jnp.float32)
        adv3 = jnp.sum(wxv3 * apack3_ref[1:2, :])
        cv3 = apack3_ref[2, 0]
        ids = jax.lax.broadcasted_iota(jnp.int32, (1, NPAD), 1)
        l2v3 = jnp.where(ids < N_NODES,
                         _leaky(asum3[0:1, :] + adv3 + cv3), -jnp.inf)
        m = jnp.max(l2v3)
        wts = jnp.exp(l2v3 - m)
        sv = jnp.sum(wts)
        aggv = jnp.dot(wts, wx3_s[...],
                       preferred_element_type=jnp.float32) / sv + pk3_ref[0:1, :]
        hv3 = jax.nn.relu(
            _ln_rows(aggv, pk3_ref[1:2, :], pk3_ref[2:3, :]) + hv2)
        jk = jnp.concatenate([hv0_ref[...], hv_ref[...], hv2, hv3], axis=1)
        gi = jnp.dot(jk, wiht_ref[...],
                     preferred_element_type=jnp.float32) + bih_ref[...]
        bhh = bhh_ref[...]
        r = jax.nn.sigmoid(gi[:, 0:HID] + bhh[:, 0:HID])
        z = jax.nn.sigmoid(gi[:, HID:2 * HID] + bhh[:, HID:2 * HID])
        n = jnp.tanh(gi[:, 2 * HID:] + r * bhh[:, 2 * HID:])
        hidden = (1.0 - z) * n
        op = jnp.dot(hidden, wop_ref[...],
                     preferred_element_type=jnp.float32) + bop_ref[...]
        rec = jax.nn.relu(_ln_rows(op, ln2_ref[0:1, :], ln2_ref[1:2, :]))
        rec = jnp.dot(rec, wrp_ref[...],
                      preferred_element_type=jnp.float32) + brp_ref[...]
        ag = jax.nn.relu(
            jnp.dot(agent_ref[...], wag_ref[...],
                    preferred_element_type=jnp.float32) + bag_ref[...])
        comb = jnp.concatenate([rec, ag], axis=1)
        value = jnp.dot(comb, wv_ref[...],
                        preferred_element_type=jnp.float32) + bv_ref[...]
        adv_q = jnp.dot(comb, wa_ref[...],
                        preferred_element_type=jnp.float32) + ba_ref[...]
        out_ref[...] = value + (adv_q - jnp.mean(adv_q, axis=1, keepdims=True))


def _k_plast(U2, WxA, SCAL, h, hv, Wxv, pk, W3, apack3, pk3, hv0, agent,
             wiht, bih, bhh, wop, bop, ln2, wrp, brp, wag, bag, wv, bv,
             wa, ba):
    full = lambda r, c: pl.BlockSpec((r, c), lambda i: (0, 0))
    return pl.pallas_call(
        _plast_body,
        grid=(N_BLKS,),
        in_specs=[
            pl.BlockSpec((2, ROW_BLK, UW), lambda i: (0, i, 0)),
            pl.BlockSpec((ROW_BLK, UW), lambda i: (i, 0)),
            pl.BlockSpec((8, ROW_BLK), lambda i: (0, i)),
            pl.BlockSpec((ROW_BLK, HID), lambda i: (i, 0)),
            full(1, HID), full(1, HID), full(8, HID),
            full(HID, HID), full(8, HID), full(8, HID),
            full(1, HID), full(1, 16),
            full(512, 384), full(1, 384), full(1, 384),
            full(HID, HID), full(1, HID), full(2, HID),
            full(HID, 512), full(1, 512), full(16, HID), full(1, HID),
            full(640, 1), full(1, 1), full(640, 8), full(1, 8),
        ],
        out_specs=pl.BlockSpec((1, 8), lambda i: (0, 0)),
        out_shape=jax.ShapeDtypeStruct((1, 8), jnp.float32),
        scratch_shapes=[
            pltpu.SMEM((1, 1), jnp.float32),
            pltpu.SMEM((1, 1), jnp.float32),
            pltpu.VMEM((1, HID), jnp.float32),
            pltpu.VMEM((NPAD, HID), jnp.float32),
            pltpu.VMEM((1, NPAD), jnp.float32),
        ],
    )(U2, WxA, SCAL, h, hv, Wxv, pk, W3, apack3, pk3, hv0, agent,
      wiht, bih, bhh, wop, bop, ln2, wrp, brp, wag, bag, wv, bv, wa, ba)


_NC = 2
_NS = 16
_NW = _NC * _NS
_EPW = N_EDGES // _NW
_EB = 80
_EBLKS = _EPW // _EB
_RPT = NPAD // _NS


_NBUF = 3


def _sc_edge_body(wxa_hbm, adst_hbm, lv_hbm, src_hbm, dst_hbm,
                  ea0_hbm, ea1_hbm, ea2_hbm, cw_hbm, out_hbm, e_v, cwb,
                  *rest):
    bufs = []
    for par in range(_NBUF):
        o = par * 13
        bufs.append(dict(
            src=rest[o + 0], dst=rest[o + 1],
            ea0=rest[o + 2], ea1=rest[o + 3], ea2=rest[o + 4],
            ta=rest[o + 5], tl=rest[o + 6], dsc=rest[o + 7],
            sr=rest[o + 8], si=rest[o + 9], srm=rest[o + 10],
            st=rest[o + 11], ss=rest[o + 12]))
    u_sh = rest[_NBUF * 13]
    cid = lax.axis_index("c")
    sid = lax.axis_index("s")
    wid = sid * _NC + cid
    ebase = wid * _EPW
    rbase = sid * _RPT
    onehot = jnp.where(lax.iota(jnp.int32, 16) == 0, 1.0, 0.0)
    col128 = jnp.full((16,), HID, jnp.int32)
    iota16 = lax.iota(jnp.int32, 16)
    zero16 = jnp.zeros((16,), jnp.float32)
    pltpu.sync_copy(cw_hbm, cwb)
    w0 = plsc.load_gather(cwb, [jnp.full((16,), 0, jnp.int32)])
    w1 = plsc.load_gather(cwb, [jnp.full((16,), 1, jnp.int32)])
    w2 = plsc.load_gather(cwb, [jnp.full((16,), 2, jnp.int32)])

    def issue_idx(g, p):
        base = ebase + g * _EB
        pltpu.async_copy(src_hbm.at[pl.ds(base, _EB)], p['src'], p['si'])
        pltpu.async_copy(dst_hbm.at[pl.ds(base, _EB)], p['dst'], p['si'])
        pltpu.async_copy(ea0_hbm.at[pl.ds(base, _EB)], p['ea0'], p['si'])
        pltpu.async_copy(ea1_hbm.at[pl.ds(base, _EB)], p['ea1'], p['si'])
        pltpu.async_copy(ea2_hbm.at[pl.ds(base, _EB)], p['ea2'], p['si'])

    def wait_idx(p):
        pltpu.make_async_copy(src_hbm.at[pl.ds(0, _EB)], p['src'], p['si']).wait()
        pltpu.make_async_copy(dst_hbm.at[pl.ds(0, _EB)], p['dst'], p['si']).wait()
        pltpu.make_async_copy(ea0_hbm.at[pl.ds(0, _EB)], p['ea0'], p['si']).wait()
        pltpu.make_async_copy(ea1_hbm.at[pl.ds(0, _EB)], p['ea1'], p['si']).wait()
        pltpu.make_async_copy(ea2_hbm.at[pl.ds(0, _EB)], p['ea2'], p['si']).wait()

    def issue_rows(p):
        pltpu.async_copy(wxa_hbm.at[p['src']], p['sr'], p['srm'])

    def wait_rows(p):
        pltpu.make_async_copy(wxa_hbm.at[p['src']], p['sr'], p['srm']).wait()

    def issue_tabs(p):
        pltpu.async_copy(adst_hbm.at[p['dst']], p['ta'], p['st'])
        pltpu.async_copy(lv_hbm.at[p['dst']], p['tl'], p['st'])

    def wait_tabs(p):
        pltpu.make_async_copy(adst_hbm.at[p['dst']], p['ta'], p['st']).wait()
        pltpu.make_async_copy(lv_hbm.at[p['dst']], p['tl'], p['st']).wait()

    def issue_scat(p):
        pltpu.async_copy(p['sr'], u_sh.at[p['dsc']], p['ss'], add=True)

    def wait_scat(p):
        pltpu.make_async_copy(p['sr'], u_sh.at[p['dsc']], p['ss']).wait()

    def compute(p):
        for j in range(_EB // 16):
            dsj = pl.ds(16 * j, 16)
            av = plsc.load_gather(p['sr'], [iota16 + (16 * j), col128])
            ce = w0 * p['ea0'][dsj] + w1 * p['ea1'][dsj] + w2 * p['ea2'][dsj]
            t = av + p['ta'][dsj] + ce
            lg = jnp.maximum(t, _NEG_SLOPE * t)
            e_v[dsj] = jnp.exp(jnp.minimum(lg - p['tl'][dsj], 60.0))
            p['dsc'][dsj] = p['dst'][dsj]

        def srow(i2, c2):
            for u in range(2):
                j = 2 * i2 + u
                ev = plsc.load_gather(e_v, [jnp.full((16,), j, jnp.int32)])
                for k in range(HID // 16):
                    p['sr'][j, pl.ds(16 * k, 16)] = (
                        p['sr'][j, pl.ds(16 * k, 16)] * ev)
                p['sr'][j, pl.ds(HID, 16)] = ev * onehot
            return c2

        lax.fori_loop(0, _EB // 2, srow, 0)

    issue_idx(0, bufs[0])
    issue_idx(1, bufs[1])

    def zrow(j, c):
        for k in range(UW // 16):
            bufs[2]['sr'][j, pl.ds(16 * k, 16)] = zero16
        return c

    lax.fori_loop(0, _EB, zrow, 0)

    def zcp(j, c):
        pltpu.sync_copy(bufs[2]['sr'], u_sh.at[pl.ds(rbase + j * _EB, _EB)])
        return c

    lax.fori_loop(0, _RPT // _EB, zcp, 0)
    plsc.subcore_barrier()
    wait_idx(bufs[0])
    issue_rows(bufs[0])
    issue_tabs(bufs[0])
    wait_idx(bufs[1])
    issue_rows(bufs[1])
    issue_tabs(bufs[1])

    def phase(g, p, q, r, has_next2):
        if has_next2:
            issue_idx(g + 2, r)
        wait_rows(p)
        wait_tabs(p)
        compute(p)
        issue_scat(p)

        @pl.when(g >= 1)
        def _():
            wait_scat(r)

        if has_next2:
            wait_idx(r)
            issue_rows(r)
            issue_tabs(r)

    def three(i, c):
        g = 3 * i
        phase(g, bufs[0], bufs[1], bufs[2], True)
        phase(g + 1, bufs[1], bufs[2], bufs[0], True)
        phase(g + 2, bufs[2], bufs[0], bufs[1], True)
        return c

    lax.fori_loop(0, (_EBLKS - 2) // 3, three, 0)
    phase(_EBLKS - 2, bufs[0], bufs[1], bufs[2], False)
    phase(_EBLKS - 1, bufs[1], bufs[2], bufs[0], False)
    wait_scat(bufs[1])
    plsc.subcore_barrier()
    pltpu.sync_copy(u_sh.at[pl.ds(rbase, _RPT)],
                    out_hbm.at[cid, pl.ds(rbase, _RPT)])


def _sc_scratch_types():
    per_par = [
        pltpu.VMEM((_EB,), jnp.int32),
        pltpu.VMEM((_EB,), jnp.int32),
        pltpu.VMEM((_EB,), jnp.float32),
        pltpu.VMEM((_EB,), jnp.float32),
        pltpu.VMEM((_EB,), jnp.float32),
        pltpu.VMEM((_EB,), jnp.float32),
        pltpu.VMEM((_EB,), jnp.float32),
        pltpu.VMEM((_EB,), jnp.int32),
        pltpu.VMEM((_EB, UW), jnp.float32),
        pltpu.SemaphoreType.DMA,
        pltpu.SemaphoreType.DMA,
        pltpu.SemaphoreType.DMA,
        pltpu.SemaphoreType.DMA,
    ]
    return ([pltpu.VMEM((_EB,), jnp.float32),
             pltpu.VMEM((16,), jnp.float32)] + per_par * _NBUF
            + [pltpu.VMEM_SHARED((NPAD, UW), jnp.float32)])


@functools.partial(
    pl.kernel,
    mesh=plsc.VectorSubcoreMesh(core_axis_name="c", subcore_axis_name="s"),
    out_type=jax.ShapeDtypeStruct((2, NPAD, UW), jnp.float32),
    compiler_params=pltpu.CompilerParams(needs_layout_passes=False,
                                         use_tc_tiling_on_sc=False),
    scratch_types=_sc_scratch_types(),
)
def _k_sc(wxa_hbm, adst_hbm, lv_hbm, src_hbm, dst_hbm,
          ea0_hbm, ea1_hbm, ea2_hbm, cw_hbm, out_hbm, *rest):
    _sc_edge_body(wxa_hbm, adst_hbm, lv_hbm, src_hbm, dst_hbm,
                  ea0_hbm, ea1_hbm, ea2_hbm, cw_hbm, out_hbm, *rest)


def _edge_phase(WxA, SCAL, src, dst, ea0, ea1, ea2, cw):
    return _k_sc(WxA, SCAL[1], SCAL[2], src, dst, ea0, ea1, ea2, cw)


def kernel(x, edge_index, edge_attr, agent_features, params):
    src = edge_index[0].astype(jnp.int32)
    dst = edge_index[1].astype(jnp.int32)
    xp = jnp.pad(x, ((0, NPAD - N_NODES), (0, 0)))

    gat = params['gat']
    weM = jnp.stack([p['W_e'] @ p['a_e'] for p in gat], axis=1)
    ea0 = edge_attr[:, 0]
    ea1 = edge_attr[:, 1]
    ea2 = edge_attr[:, 2]
    cw = jnp.pad(weM.T, ((0, 0), (0, 13)))

    def apack_of(p, l):
        cv = 0.5 * weM[0, l]
        return jnp.concatenate([
            p['a_src'][None, :], p['a_dst'][None, :],
            jnp.full((1, HID), cv, jnp.float32),
            jnp.zeros((5, HID), jnp.float32)], axis=0)

    def pk_of(p):
        return jnp.concatenate([
            p['b'][None, :], p['ln_g'][None, :], p['ln_b'][None, :],
            jnp.zeros((5, HID), jnp.float32)], axis=0)

    p0, p1, p2 = gat[0], gat[1], gat[2]
    h, WxA0, SCAL0, hv0, Wxv0 = _k_preatt(
        xp, params['W_ne'], params['b_ne'][None, :],
        params['W_vn'], params['b_vn'][None, :],
        p0['W'], apack_of(p0, 0), p0['a_src'][:, None])
    U2 = _edge_phase(WxA0, SCAL0, src, dst, ea0, ea1, ea2, cw[0])
    h1, WxA1, SCAL1, hv1, Wxv1 = _k_fuse(
        U2, WxA0, SCAL0, h, hv0, Wxv0, pk_of(p0),
        p1['W'], apack_of(p1, 1), p1['a_src'][:, None])
    U2b = _edge_phase(WxA1, SCAL1, src, dst, ea0, ea1, ea2, cw[1])
    out = _k_plast(
        U2b, WxA1, SCAL1, h1, hv1, Wxv1, pk_of(p1),
        p2['W'], apack_of(p2, 2), pk_of(p2), hv0, agent_features,
        params['W_ih'].T, params['b_ih'][None, :], params['b_hh'][None, :],
        params['W_op'], params['b_op'][None, :],
        jnp.stack([params['ln2_g'], params['ln2_b']], axis=0),
        params['W_rp'], params['b_rp'][None, :],
        params['W_ag'], params['b_ag'][None, :],
        params['W_v'], params['b_v'][None, :],
        params['W_a'], params['b_a'][None, :])
    return out
